# Initial kernel scaffold; baseline (speedup 1.0000x reference)
import jax
import jax.numpy as jnp
from jax.experimental import pallas as pl


def kernel(protein_pos, protein_v, batch_protein, init_ligand_pos, init_ligand_v, batch_ligand, time_step, edge_index, W_protein, b_protein, W_ligand, b_ligand, edge_w1, edge_b1, edge_w2, edge_b2, node_w1, node_b1, node_w2, node_b2, coord_w, coord_b, v_out_w, v_out_b):
    raise NotImplementedError("write your pallas kernel here")


# speedup vs baseline: 4.4711x; 4.4711x over previous
import functools
import math

import jax
import jax.numpy as jnp
from jax import lax
from jax.experimental import pallas as pl
from jax.experimental.pallas import tpu as pltpu
from jax.experimental.pallas import tpu_sc as plsc

N_PROT = 8000
N_LIG = 2000
N_TOT = 10000
N_EDGE = 320000
NUM_GRAPHS = 16
HID = 128
NUM_RBF = 16
R_MAX = 10.0
NUM_LAYERS = 2
NUM_CLASSES = 13

NC = 2
NS = 16
NW = NC * NS
EB = 128
NB = 80
EW = NB * EB
E_PAD = NW * EW
NP_PAD = 10240
DPF = NP_PAD * 4
ROWS_PER_TILE = NP_PAD // NS
DPF_PER_TILE = DPF // NS


@functools.cache
def _mesh():
    return plsc.VectorSubcoreMesh(
        core_axis_name="c", subcore_axis_name="s",
        num_cores=NC, num_subcores=NS)


def _pre_body(ppos, pv, bp_col, bp_row, lpos, lv, bl_col, bl_row, t_col,
              wp, bpb, wlv, wlt, blb, w1a, w1b, b1,
              h_out, pos_out, offl_out, ta_out, tb_out):
    f32 = jnp.float32
    ppos_v = ppos[...]
    lpos_v = lpos[...]
    ohp_t = (lax.broadcasted_iota(jnp.int32, (NUM_GRAPHS, N_PROT), 0)
             == bp_row[...]).astype(f32)
    sums = jnp.dot(ohp_t, ppos_v)
    cnt = jnp.sum(ohp_t, axis=1, keepdims=True)
    mean = sums / jnp.maximum(cnt, 1.0)
    ohp = (lax.broadcasted_iota(jnp.int32, (N_PROT, NUM_GRAPHS), 1)
           == bp_col[...]).astype(f32)
    ohl = (lax.broadcasted_iota(jnp.int32, (N_LIG, NUM_GRAPHS), 1)
           == bl_col[...]).astype(f32)
    p_c = ppos_v - jnp.dot(ohp, mean)
    offl = jnp.dot(ohl, mean)
    l_c = lpos_v - offl
    half = 4
    freq = jnp.exp(lax.broadcasted_iota(jnp.int32, (1, half), 1).astype(f32)
                   * (-math.log(10000.0) / (half - 1)))
    ang = t_col[...] * freq
    te = jnp.concatenate([jnp.sin(ang), jnp.cos(ang)], axis=1)
    te_l = jnp.dot(ohl, te)
    h_p = jnp.dot(pv[...], wp[...]) + bpb[...]
    h_l = jnp.dot(lv[...], wlv[...]) + jnp.dot(te_l, wlt[...]) + blb[...]
    h = jnp.concatenate([h_p, h_l], axis=0)
    pos = jnp.concatenate([p_c, l_c], axis=0)
    h_out[...] = h
    pos_out[...] = pos
    offl_out[...] = offl
    ta_out[...] = jnp.dot(h, w1a[...]) + b1[...]
    tb_out[...] = jnp.dot(h, w1b[...])


def _run_pre(ppos4, pv, bp_col, bp_row, lpos4, lv, bl_col, bl_row, t_col,
             wp, bpb, wlv, wlt, blb, w1a, w1b, b1):
    f32 = jnp.float32
    return pl.pallas_call(
        _pre_body,
        out_shape=(
            jax.ShapeDtypeStruct((N_TOT, HID), f32),
            jax.ShapeDtypeStruct((N_TOT, 4), f32),
            jax.ShapeDtypeStruct((N_LIG, 4), f32),
            jax.ShapeDtypeStruct((N_TOT, HID), f32),
            jax.ShapeDtypeStruct((N_TOT, HID), f32),
        ),
    )(ppos4, pv, bp_col, bp_row, lpos4, lv, bl_col, bl_row, t_col,
      wp, bpb, wlv, wlt, blb, w1a, w1b, b1)


def _gather_body(ta_hbm, tb_hbm, posp_hbm, srcm_hbm, dstm_hbm,
                 as_out, bd_out, rx_out, ry_out, rz_out,
                 idx_s, idx_d, posv, rows_a, rows_b, rxb, ryb, rzb, sem):
    cid = lax.axis_index("c")
    sid = lax.axis_index("s")
    wid = cid * NS + sid
    pltpu.sync_copy(srcm_hbm.at[pl.ds(wid * NB, NB)], idx_s)
    pltpu.sync_copy(dstm_hbm.at[pl.ds(wid * NB, NB)], idx_d)
    pltpu.sync_copy(posp_hbm, posv)

    def body(j, carry):
        base = wid * EW + j * EB
        da = pltpu.async_copy(ta_hbm.at[idx_s.at[j]], rows_a, sem)
        db = pltpu.async_copy(tb_hbm.at[idx_d.at[j]], rows_b, sem)

        def grp(gi, c2):
            s16 = idx_s[j, pl.ds(gi * 16, 16)]
            d16 = idx_d[j, pl.ds(gi * 16, 16)]
            fs = lax.shift_left(s16, 2)
            fd = lax.shift_left(d16, 2)
            for c, buf in ((0, rxb), (1, ryb), (2, rzb)):
                ps = plsc.load_gather(posv, [fs + c])
                pd = plsc.load_gather(posv, [fd + c])
                buf[pl.ds(gi * 16, 16)] = pd - ps
            return c2

        lax.fori_loop(0, EB // 16, grp, 0)
        da.wait()
        db.wait()
        pltpu.sync_copy(rows_a, as_out.at[pl.ds(base, EB)])
        pltpu.sync_copy(rows_b, bd_out.at[pl.ds(base, EB)])
        pltpu.sync_copy(rxb, rx_out.at[pl.ds(base, EB)])
        pltpu.sync_copy(ryb, ry_out.at[pl.ds(base, EB)])
        pltpu.sync_copy(rzb, rz_out.at[pl.ds(base, EB)])
        return carry

    lax.fori_loop(0, NB, body, 0)


def _run_gather(ta, tb, posp, srcm, dstm):
    f32 = jnp.float32
    return pl.kernel(
        _gather_body,
        out_type=(
            jax.ShapeDtypeStruct((E_PAD, HID), f32),
            jax.ShapeDtypeStruct((E_PAD, HID), f32),
            jax.ShapeDtypeStruct((E_PAD,), f32),
            jax.ShapeDtypeStruct((E_PAD,), f32),
            jax.ShapeDtypeStruct((E_PAD,), f32),
        ),
        mesh=_mesh(),
        compiler_params=pltpu.CompilerParams(needs_layout_passes=False),
        scratch_types=[
            pltpu.VMEM((NB, EB), jnp.int32),
            pltpu.VMEM((NB, EB), jnp.int32),
            pltpu.VMEM((NP_PAD * 4,), f32),
            pltpu.VMEM((EB, HID), f32),
            pltpu.VMEM((EB, HID), f32),
            pltpu.VMEM((EB,), f32),
            pltpu.VMEM((EB,), f32),
            pltpu.VMEM((EB,), f32),
            pltpu.SemaphoreType.DMA,
        ],
    )(ta, tb, posp, srcm, dstm)


_EBLK = 1024


def _edge_body(asrc, bdst, rx, ry, rz, w1c, w2, b2, cw, cb,
               m_out, wx_out, wy_out, wz_out):
    f32 = jnp.float32
    i = pl.program_id(0)
    rxv = rx[...]
    ryv = ry[...]
    rzv = rz[...]
    d_flat = jnp.sqrt(rxv * rxv + ryv * ryv + rzv * rzv)
    d_row = d_flat.reshape(1, _EBLK)
    delta = R_MAX / (NUM_RBF - 1)
    cent_col = (lax.broadcasted_iota(jnp.int32, (NUM_RBF, 1), 0)
                .astype(f32) * delta)
    rbf_t = jnp.exp((-0.5 / (delta * delta)) * (d_row - cent_col) ** 2)
    contrib = lax.dot_general(rbf_t, w1c[...],
                              (((0,), (0,)), ((), ())))
    m1 = jnp.maximum(asrc[...] + bdst[...] + contrib, 0.0)
    m = jnp.maximum(jnp.dot(m1, w2[...]) + b2[...], 0.0)
    cw_row = lax.dot_general(cw[...], m,
                             (((0,), (1,)), ((), ())))
    w_row = (cw_row + cb[...]) / (d_row + 1.0)
    w_flat = jnp.squeeze(w_row, axis=0)
    gid_col = i * _EBLK + lax.broadcasted_iota(jnp.int32, (_EBLK, 1), 0)
    gid_flat = i * _EBLK + lax.broadcasted_iota(jnp.int32, (_EBLK,), 0)
    m_out[...] = jnp.where(gid_col < N_EDGE, m, 0.0)
    vf = gid_flat < N_EDGE
    wx_out[...] = jnp.where(vf, rxv * w_flat, 0.0)
    wy_out[...] = jnp.where(vf, ryv * w_flat, 0.0)
    wz_out[...] = jnp.where(vf, rzv * w_flat, 0.0)


def _run_edge(asrc, bdst, rx, ry, rz, w1c, w2, b2, cw, cb):
    f32 = jnp.float32
    nblk = E_PAD // _EBLK
    full = lambda i: (0, 0)
    return pl.pallas_call(
        _edge_body,
        grid=(nblk,),
        in_specs=[
            pl.BlockSpec((_EBLK, HID), lambda i: (i, 0)),
            pl.BlockSpec((_EBLK, HID), lambda i: (i, 0)),
            pl.BlockSpec((_EBLK,), lambda i: (i,)),
            pl.BlockSpec((_EBLK,), lambda i: (i,)),
            pl.BlockSpec((_EBLK,), lambda i: (i,)),
            pl.BlockSpec((NUM_RBF, HID), full),
            pl.BlockSpec((HID, HID), full),
            pl.BlockSpec((1, HID), full),
            pl.BlockSpec((HID, 1), full),
            pl.BlockSpec((1, 1), full),
        ],
        out_specs=[
            pl.BlockSpec((_EBLK, HID), lambda i: (i, 0)),
            pl.BlockSpec((_EBLK,), lambda i: (i,)),
            pl.BlockSpec((_EBLK,), lambda i: (i,)),
            pl.BlockSpec((_EBLK,), lambda i: (i,)),
        ],
        out_shape=(
            jax.ShapeDtypeStruct((E_PAD, HID), f32),
            jax.ShapeDtypeStruct((E_PAD,), f32),
            jax.ShapeDtypeStruct((E_PAD,), f32),
            jax.ShapeDtypeStruct((E_PAD,), f32),
        ),
    )(asrc, bdst, rx, ry, rz, w1c, w2, b2, cw, cb)


def _scatter_body(m_hbm, wx_hbm, wy_hbm, wz_hbm, dstm_hbm, zagg_hbm, zdp_hbm,
                  agg_out, dpf_out,
                  idx_d, mrows, wxb, wyb, wzb, fix, fiy, fiz,
                  sh_agg, sh_dpf, sem):
    cid = lax.axis_index("c")
    sid = lax.axis_index("s")
    wid = cid * NS + sid
    r0 = sid * ROWS_PER_TILE
    f0 = sid * DPF_PER_TILE
    pltpu.sync_copy(zagg_hbm.at[pl.ds(r0, ROWS_PER_TILE)],
                    sh_agg.at[pl.ds(r0, ROWS_PER_TILE)])
    pltpu.sync_copy(zdp_hbm.at[pl.ds(f0, DPF_PER_TILE)],
                    sh_dpf.at[pl.ds(f0, DPF_PER_TILE)])
    pltpu.sync_copy(dstm_hbm.at[pl.ds(wid * NB, NB)], idx_d)
    plsc.subcore_barrier()

    def body(j, carry):
        base = wid * EW + j * EB
        dm = pltpu.async_copy(m_hbm.at[pl.ds(base, EB)], mrows, sem)
        dx = pltpu.async_copy(wx_hbm.at[pl.ds(base, EB)], wxb, sem)
        dy = pltpu.async_copy(wy_hbm.at[pl.ds(base, EB)], wyb, sem)
        dz = pltpu.async_copy(wz_hbm.at[pl.ds(base, EB)], wzb, sem)

        def grp(gi, c2):
            d16 = idx_d[j, pl.ds(gi * 16, 16)]
            f = lax.shift_left(d16, 2)
            fix[pl.ds(gi * 16, 16)] = f
            fiy[pl.ds(gi * 16, 16)] = f + 1
            fiz[pl.ds(gi * 16, 16)] = f + 2
            return c2

        lax.fori_loop(0, EB // 16, grp, 0)
        dm.wait()
        dx.wait()
        dy.wait()
        dz.wait()
        pltpu.sync_copy(mrows, sh_agg.at[idx_d.at[j]], add=True)
        pltpu.sync_copy(wxb, sh_dpf.at[fix], add=True)
        pltpu.sync_copy(wyb, sh_dpf.at[fiy], add=True)
        pltpu.sync_copy(wzb, sh_dpf.at[fiz], add=True)
        return carry

    lax.fori_loop(0, NB, body, 0)
    plsc.subcore_barrier()
    pltpu.sync_copy(sh_agg.at[pl.ds(r0, ROWS_PER_TILE)],
                    agg_out.at[cid, pl.ds(r0, ROWS_PER_TILE)])
    pltpu.sync_copy(sh_dpf.at[pl.ds(f0, DPF_PER_TILE)],
                    dpf_out.at[cid, pl.ds(f0, DPF_PER_TILE)])


def _run_scatter(m, wx, wy, wz, dstm, zagg, zdp):
    f32 = jnp.float32
    return pl.kernel(
        _scatter_body,
        out_type=(
            jax.ShapeDtypeStruct((NC, NP_PAD, HID), f32),
            jax.ShapeDtypeStruct((NC, DPF), f32),
        ),
        mesh=_mesh(),
        compiler_params=pltpu.CompilerParams(needs_layout_passes=False),
        scratch_types=[
            pltpu.VMEM((NB, EB), jnp.int32),
            pltpu.VMEM((EB, HID), f32),
            pltpu.VMEM((EB,), f32),
            pltpu.VMEM((EB,), f32),
            pltpu.VMEM((EB,), f32),
            pltpu.VMEM((EB,), jnp.int32),
            pltpu.VMEM((EB,), jnp.int32),
            pltpu.VMEM((EB,), jnp.int32),
            pltpu.VMEM_SHARED((NP_PAD, HID), f32),
            pltpu.VMEM_SHARED((DPF,), f32),
            pltpu.SemaphoreType.DMA,
        ],
    )(m, wx, wy, wz, dstm, zagg, zdp)


def _node_body(h, pos, agg2, dp2, w1h, w1g, b1n, w2n, b2n, ew1a, ew1b, eb1,
               h_out, pos_out, ta_out, tb_out):
    f32 = jnp.float32
    hv = h[...]
    a2 = agg2[...]
    agg = a2[0, :N_TOT] + a2[1, :N_TOT]
    u = jnp.maximum(jnp.dot(hv, w1h[...]) + jnp.dot(agg, w1g[...]) + b1n[...],
                    0.0)
    hn = hv + jnp.dot(u, w2n[...]) + b2n[...]
    d2v = dp2[...]
    dp = d2v[0] + d2v[1]
    mask = (lax.broadcasted_iota(jnp.int32, (N_TOT, 1), 0)
            >= N_PROT).astype(f32)
    posn = pos[...] + dp * mask
    h_out[...] = hn
    pos_out[...] = posn
    ta_out[...] = jnp.dot(hn, ew1a[...]) + eb1[...]
    tb_out[...] = jnp.dot(hn, ew1b[...])


def _run_node(h, pos, agg2, dp2, w1h, w1g, b1n, w2n, b2n, ew1a, ew1b, eb1):
    f32 = jnp.float32
    return pl.pallas_call(
        _node_body,
        out_shape=(
            jax.ShapeDtypeStruct((N_TOT, HID), f32),
            jax.ShapeDtypeStruct((N_TOT, 4), f32),
            jax.ShapeDtypeStruct((N_TOT, HID), f32),
            jax.ShapeDtypeStruct((N_TOT, HID), f32),
        ),
    )(h, pos, agg2, dp2, w1h, w1g, b1n, w2n, b2n, ew1a, ew1b, eb1)


def _final_body(hl, posl, offl, vw, vb, pos_out, v_out):
    pos_out[...] = posl[...] + offl[...]
    v_out[...] = jnp.dot(hl[...], vw[...]) + vb[...]


def _run_final(hl, posl, offl, vw, vb):
    f32 = jnp.float32
    return pl.pallas_call(
        _final_body,
        out_shape=(
            jax.ShapeDtypeStruct((N_LIG, 4), f32),
            jax.ShapeDtypeStruct((N_LIG, NUM_CLASSES), f32),
        ),
    )(hl, posl, offl, vw, vb)


def _pack_pos(pos):
    return jnp.reshape(jnp.pad(pos, ((0, NP_PAD - N_TOT), (0, 0))),
                       (NP_PAD * 4,))


def kernel(protein_pos, protein_v, batch_protein, init_ligand_pos,
           init_ligand_v, batch_ligand, time_step, edge_index,
           W_protein, b_protein, W_ligand, b_ligand,
           edge_w1, edge_b1, edge_w2, edge_b2,
           node_w1, node_b1, node_w2, node_b2,
           coord_w, coord_b, v_out_w, v_out_b):
    f32 = jnp.float32
    i32 = jnp.int32

    ppos4 = jnp.pad(protein_pos.astype(f32), ((0, 0), (0, 1)))
    lpos4 = jnp.pad(init_ligand_pos.astype(f32), ((0, 0), (0, 1)))
    bp_col = batch_protein.astype(i32).reshape(N_PROT, 1)
    bp_row = batch_protein.astype(i32).reshape(1, N_PROT)
    bl_col = batch_ligand.astype(i32).reshape(N_LIG, 1)
    bl_row = batch_ligand.astype(i32).reshape(1, N_LIG)
    t_col = time_step.astype(f32).reshape(NUM_GRAPHS, 1)

    wp = jnp.pad(W_protein, ((0, 0), (0, 1)))
    bpb = jnp.pad(b_protein, (0, 1)).reshape(1, HID)
    wlv = jnp.pad(W_ligand[:NUM_CLASSES], ((0, 0), (0, 1)))
    wlt = jnp.pad(W_ligand[NUM_CLASSES:], ((0, 0), (0, 1)))
    blb = jnp.concatenate([b_ligand, jnp.ones((1,), f32)]).reshape(1, HID)
    w1a = edge_w1[:HID]
    w1b = edge_w1[HID:2 * HID]
    w1c = edge_w1[2 * HID:]
    b1 = edge_b1.reshape(1, HID)
    b2 = edge_b2.reshape(1, HID)
    cb = coord_b.reshape(1, 1)
    w1h = node_w1[:HID]
    w1g = node_w1[HID:]
    b1n = node_b1.reshape(1, HID)
    b2n = node_b2.reshape(1, HID)
    vb = v_out_b.reshape(1, NUM_CLASSES)

    src = edge_index[0].astype(i32)
    dst = edge_index[1].astype(i32)
    pad_idx = (jnp.arange(E_PAD - N_EDGE, dtype=i32)) % N_TOT
    srcm = jnp.concatenate([src, pad_idx]).reshape(NW * NB, EB)
    dstm = jnp.concatenate([dst, pad_idx]).reshape(NW * NB, EB)

    zagg = jnp.zeros((NP_PAD, HID), f32)
    zdp = jnp.zeros((DPF,), f32)

    h, pos, offl, ta, tb = _run_pre(
        ppos4, protein_v, bp_col, bp_row, lpos4, init_ligand_v, bl_col,
        bl_row, t_col, wp, bpb, wlv, wlt, blb, w1a, w1b, b1)

    for _ in range(NUM_LAYERS):
        posp = _pack_pos(pos)
        asrc, bdst, rx, ry, rz = _run_gather(ta, tb, posp, srcm, dstm)
        m, wx, wy, wz = _run_edge(asrc, bdst, rx, ry, rz, w1c, edge_w2, b2,
                                  coord_w, cb)
        agg2, dpf2 = _run_scatter(m, wx, wy, wz, dstm, zagg, zdp)
        dp2 = dpf2.reshape(NC, NP_PAD, 4)[:, :N_TOT]
        h, pos, ta, tb = _run_node(
            h, pos, agg2, dp2, w1h, w1g, b1n, node_w2, b2n, w1a, w1b, b1)

    posl4, pred_v = _run_final(
        h[N_PROT:], pos[N_PROT:], offl, v_out_w, vb)
    return posl4[:, :3], pred_v


# speedup vs baseline: 5.0712x; 1.1342x over previous
import functools
import math

import jax
import jax.numpy as jnp
from jax import lax
from jax.experimental import pallas as pl
from jax.experimental.pallas import tpu as pltpu
from jax.experimental.pallas import tpu_sc as plsc

N_PROT = 8000
N_LIG = 2000
N_TOT = 10000
N_EDGE = 320000
NUM_GRAPHS = 16
HID = 128
NUM_RBF = 16
R_MAX = 10.0
NUM_LAYERS = 2
NUM_CLASSES = 13

NC = 2
NS = 16
NW = NC * NS
EB = 128
NB = 80
EW = NB * EB
E_PAD = NW * EW
NP_PAD = 10240
DPF = NP_PAD * 4
ROWS_PER_TILE = NP_PAD // NS
DPF_PER_TILE = DPF // NS


@functools.cache
def _mesh():
    return plsc.VectorSubcoreMesh(
        core_axis_name="c", subcore_axis_name="s",
        num_cores=NC, num_subcores=NS)


def _pre_body(ppos, pv, bp_col, bp_row, lpos, lv, bl_col, bl_row, t_col,
              wp, bpb, wlv, wlt, blb, w1a, w1b, b1,
              h_out, pos_out, offl_out, ta_out, tb_out):
    f32 = jnp.float32
    ppos_v = ppos[...]
    lpos_v = lpos[...]
    ohp_t = (lax.broadcasted_iota(jnp.int32, (NUM_GRAPHS, N_PROT), 0)
             == bp_row[...]).astype(f32)
    sums = jnp.dot(ohp_t, ppos_v)
    cnt = jnp.sum(ohp_t, axis=1, keepdims=True)
    mean = sums / jnp.maximum(cnt, 1.0)
    ohp = (lax.broadcasted_iota(jnp.int32, (N_PROT, NUM_GRAPHS), 1)
           == bp_col[...]).astype(f32)
    ohl = (lax.broadcasted_iota(jnp.int32, (N_LIG, NUM_GRAPHS), 1)
           == bl_col[...]).astype(f32)
    p_c = ppos_v - jnp.dot(ohp, mean)
    offl = jnp.dot(ohl, mean)
    l_c = lpos_v - offl
    half = 4
    freq = jnp.exp(lax.broadcasted_iota(jnp.int32, (1, half), 1).astype(f32)
                   * (-math.log(10000.0) / (half - 1)))
    ang = t_col[...] * freq
    te = jnp.concatenate([jnp.sin(ang), jnp.cos(ang)], axis=1)
    te_l = jnp.dot(ohl, te)
    h_p = jnp.dot(pv[...], wp[...]) + bpb[...]
    h_l = jnp.dot(lv[...], wlv[...]) + jnp.dot(te_l, wlt[...]) + blb[...]
    h = jnp.concatenate([h_p, h_l], axis=0)
    pos = jnp.concatenate([p_c, l_c], axis=0)
    h_out[...] = h
    pos_out[...] = pos
    offl_out[...] = offl
    ta_out[...] = jnp.dot(h, w1a[...]) + b1[...]
    tb_out[...] = jnp.dot(h, w1b[...])


def _run_pre(ppos4, pv, bp_col, bp_row, lpos4, lv, bl_col, bl_row, t_col,
             wp, bpb, wlv, wlt, blb, w1a, w1b, b1):
    f32 = jnp.float32
    return pl.pallas_call(
        _pre_body,
        out_shape=(
            jax.ShapeDtypeStruct((N_TOT, HID), f32),
            jax.ShapeDtypeStruct((N_TOT, 4), f32),
            jax.ShapeDtypeStruct((N_LIG, 4), f32),
            jax.ShapeDtypeStruct((N_TOT, HID), f32),
            jax.ShapeDtypeStruct((N_TOT, HID), f32),
        ),
    )(ppos4, pv, bp_col, bp_row, lpos4, lv, bl_col, bl_row, t_col,
      wp, bpb, wlv, wlt, blb, w1a, w1b, b1)


def _gather_body(ta_hbm, tb_hbm, posp_hbm, srcm_hbm, dstm_hbm,
                 as_out, rx_out, ry_out, rz_out,
                 idx_s, idx_d, posv,
                 ra0, rb0, rx0, ry0, rz0,
                 ra1, rb1, rx1, ry1, rz1,
                 gsem0, gsem1, osem0, osem1):
    cid = lax.axis_index("c")
    sid = lax.axis_index("s")
    wid = cid * NS + sid
    pltpu.sync_copy(srcm_hbm.at[pl.ds(wid * NB, NB)], idx_s)
    pltpu.sync_copy(dstm_hbm.at[pl.ds(wid * NB, NB)], idx_d)
    pltpu.sync_copy(posp_hbm, posv)

    sets = ((ra0, rb0, rx0, ry0, rz0, gsem0, osem0),
            (ra1, rb1, rx1, ry1, rz1, gsem1, osem1))

    def start(j, p):
        ra, rb = sets[p][0], sets[p][1]
        gsem = sets[p][5]
        pltpu.async_copy(ta_hbm.at[idx_s.at[j]], ra, gsem)
        pltpu.async_copy(tb_hbm.at[idx_d.at[j]], rb, gsem)

    def drain_out(j, p):
        ra, _, rxb, ryb, rzb, _, osem = sets[p]
        base = wid * EW + j * EB
        pltpu.make_async_copy(ra, as_out.at[pl.ds(base, EB)], osem).wait()
        pltpu.make_async_copy(rxb, rx_out.at[pl.ds(base, EB)], osem).wait()
        pltpu.make_async_copy(ryb, ry_out.at[pl.ds(base, EB)], osem).wait()
        pltpu.make_async_copy(rzb, rz_out.at[pl.ds(base, EB)], osem).wait()

    def process(j, p):
        ra, rb, rxb, ryb, rzb, gsem, osem = sets[p]

        def grp(gi, c2):
            s16 = idx_s[j, pl.ds(gi * 16, 16)]
            d16 = idx_d[j, pl.ds(gi * 16, 16)]
            fs = lax.shift_left(s16, 2)
            fd = lax.shift_left(d16, 2)
            for c, buf in ((0, rxb), (1, ryb), (2, rzb)):
                ps = plsc.load_gather(posv, [fs + c])
                pd = plsc.load_gather(posv, [fd + c])
                buf[pl.ds(gi * 16, 16)] = pd - ps
            return c2

        lax.fori_loop(0, EB // 16, grp, 0)
        pltpu.make_async_copy(ta_hbm.at[idx_s.at[j]], ra, gsem).wait()
        pltpu.make_async_copy(tb_hbm.at[idx_d.at[j]], rb, gsem).wait()

        @plsc.parallel_loop(0, EB * HID // 16, unroll=8)
        def _add(i):
            r = lax.shift_right_logical(i, 3)
            g = lax.bitwise_and(i, 7)
            ra[r, pl.ds(g * 16, 16)] += rb[r, pl.ds(g * 16, 16)]

        base = wid * EW + j * EB
        pltpu.async_copy(ra, as_out.at[pl.ds(base, EB)], osem)
        pltpu.async_copy(rxb, rx_out.at[pl.ds(base, EB)], osem)
        pltpu.async_copy(ryb, ry_out.at[pl.ds(base, EB)], osem)
        pltpu.async_copy(rzb, rz_out.at[pl.ds(base, EB)], osem)

    start(0, 0)

    def body(k, carry):
        j0 = 2 * k
        j1 = j0 + 1
        process(j0, 0)

        @pl.when(k > 0)
        def _():
            drain_out(j1 - 2, 1)

        start(j1, 1)
        process(j1, 1)
        drain_out(j0, 0)

        @pl.when(k < NB // 2 - 1)
        def _():
            start(j0 + 2, 0)

        return carry

    lax.fori_loop(0, NB // 2, body, 0)
    drain_out(NB - 1, 1)


def _run_gather(ta, tb, posp, srcm, dstm):
    f32 = jnp.float32
    return pl.kernel(
        _gather_body,
        out_type=(
            jax.ShapeDtypeStruct((E_PAD, HID), f32),
            jax.ShapeDtypeStruct((E_PAD,), f32),
            jax.ShapeDtypeStruct((E_PAD,), f32),
            jax.ShapeDtypeStruct((E_PAD,), f32),
        ),
        mesh=_mesh(),
        compiler_params=pltpu.CompilerParams(needs_layout_passes=False),
        scratch_types=[
            pltpu.VMEM((NB, EB), jnp.int32),
            pltpu.VMEM((NB, EB), jnp.int32),
            pltpu.VMEM((NP_PAD * 4,), f32),
            pltpu.VMEM((EB, HID), f32),
            pltpu.VMEM((EB, HID), f32),
            pltpu.VMEM((EB,), f32),
            pltpu.VMEM((EB,), f32),
            pltpu.VMEM((EB,), f32),
            pltpu.VMEM((EB, HID), f32),
            pltpu.VMEM((EB, HID), f32),
            pltpu.VMEM((EB,), f32),
            pltpu.VMEM((EB,), f32),
            pltpu.VMEM((EB,), f32),
            pltpu.SemaphoreType.DMA,
            pltpu.SemaphoreType.DMA,
            pltpu.SemaphoreType.DMA,
            pltpu.SemaphoreType.DMA,
        ],
    )(ta, tb, posp, srcm, dstm)


_EBLK = 1024


def _edge_body(asum, rx, ry, rz, w1c, w2, b2, cw, cb,
               m_out, wx_out, wy_out, wz_out):
    f32 = jnp.float32
    i = pl.program_id(0)
    rxv = rx[...]
    ryv = ry[...]
    rzv = rz[...]
    d_flat = jnp.sqrt(rxv * rxv + ryv * ryv + rzv * rzv)
    d_row = d_flat.reshape(1, _EBLK)
    delta = R_MAX / (NUM_RBF - 1)
    cent_col = (lax.broadcasted_iota(jnp.int32, (NUM_RBF, 1), 0)
                .astype(f32) * delta)
    rbf_t = jnp.exp((-0.5 / (delta * delta)) * (d_row - cent_col) ** 2)
    contrib = lax.dot_general(rbf_t, w1c[...],
                              (((0,), (0,)), ((), ())))
    m1 = jnp.maximum(asum[...] + contrib, 0.0)
    m = jnp.maximum(jnp.dot(m1, w2[...]) + b2[...], 0.0)
    cw_row = lax.dot_general(cw[...], m,
                             (((0,), (1,)), ((), ())))
    w_row = (cw_row + cb[...]) / (d_row + 1.0)
    w_flat = jnp.squeeze(w_row, axis=0)
    gid_col = i * _EBLK + lax.broadcasted_iota(jnp.int32, (_EBLK, 1), 0)
    gid_flat = i * _EBLK + lax.broadcasted_iota(jnp.int32, (_EBLK,), 0)
    m_out[...] = jnp.where(gid_col < N_EDGE, m, 0.0)
    vf = gid_flat < N_EDGE
    wx_out[...] = jnp.where(vf, rxv * w_flat, 0.0)
    wy_out[...] = jnp.where(vf, ryv * w_flat, 0.0)
    wz_out[...] = jnp.where(vf, rzv * w_flat, 0.0)


def _run_edge(asum, rx, ry, rz, w1c, w2, b2, cw, cb):
    f32 = jnp.float32
    nblk = E_PAD // _EBLK
    full = lambda i: (0, 0)
    return pl.pallas_call(
        _edge_body,
        grid=(nblk,),
        in_specs=[
            pl.BlockSpec((_EBLK, HID), lambda i: (i, 0)),
            pl.BlockSpec((_EBLK,), lambda i: (i,)),
            pl.BlockSpec((_EBLK,), lambda i: (i,)),
            pl.BlockSpec((_EBLK,), lambda i: (i,)),
            pl.BlockSpec((NUM_RBF, HID), full),
            pl.BlockSpec((HID, HID), full),
            pl.BlockSpec((1, HID), full),
            pl.BlockSpec((HID, 1), full),
            pl.BlockSpec((1, 1), full),
        ],
        out_specs=[
            pl.BlockSpec((_EBLK, HID), lambda i: (i, 0)),
            pl.BlockSpec((_EBLK,), lambda i: (i,)),
            pl.BlockSpec((_EBLK,), lambda i: (i,)),
            pl.BlockSpec((_EBLK,), lambda i: (i,)),
        ],
        out_shape=(
            jax.ShapeDtypeStruct((E_PAD, HID), f32),
            jax.ShapeDtypeStruct((E_PAD,), f32),
            jax.ShapeDtypeStruct((E_PAD,), f32),
            jax.ShapeDtypeStruct((E_PAD,), f32),
        ),
    )(asum, rx, ry, rz, w1c, w2, b2, cw, cb)


def _scatter_body(m_hbm, wx_hbm, wy_hbm, wz_hbm, dstm_hbm, zagg_hbm, zdp_hbm,
                  agg_out, dpf_out,
                  idx_d, mr0, xb0, yb0, zb0, mr1, xb1, yb1, zb1,
                  fix, fiy, fiz, sh_agg, sh_dpf, sem0, sem1):
    cid = lax.axis_index("c")
    sid = lax.axis_index("s")
    wid = cid * NS + sid
    r0 = sid * ROWS_PER_TILE
    f0 = sid * DPF_PER_TILE
    pltpu.sync_copy(zagg_hbm.at[pl.ds(r0, ROWS_PER_TILE)],
                    sh_agg.at[pl.ds(r0, ROWS_PER_TILE)])
    pltpu.sync_copy(zdp_hbm.at[pl.ds(f0, DPF_PER_TILE)],
                    sh_dpf.at[pl.ds(f0, DPF_PER_TILE)])
    pltpu.sync_copy(dstm_hbm.at[pl.ds(wid * NB, NB)], idx_d)
    plsc.subcore_barrier()

    sets = ((mr0, xb0, yb0, zb0, sem0), (mr1, xb1, yb1, zb1, sem1))

    def start(j, p):
        mb, xb, yb, zb, sem = sets[p]
        base = wid * EW + j * EB
        pltpu.async_copy(m_hbm.at[pl.ds(base, EB)], mb, sem)
        pltpu.async_copy(wx_hbm.at[pl.ds(base, EB)], xb, sem)
        pltpu.async_copy(wy_hbm.at[pl.ds(base, EB)], yb, sem)
        pltpu.async_copy(wz_hbm.at[pl.ds(base, EB)], zb, sem)

    def process(j, p):
        mb, xb, yb, zb, sem = sets[p]

        def grp(gi, c2):
            d16 = idx_d[j, pl.ds(gi * 16, 16)]
            f = lax.shift_left(d16, 2)
            fix[pl.ds(gi * 16, 16)] = f
            fiy[pl.ds(gi * 16, 16)] = f + 1
            fiz[pl.ds(gi * 16, 16)] = f + 2
            return c2

        lax.fori_loop(0, EB // 16, grp, 0)
        base = wid * EW + j * EB
        pltpu.make_async_copy(m_hbm.at[pl.ds(base, EB)], mb, sem).wait()
        pltpu.make_async_copy(wx_hbm.at[pl.ds(base, EB)], xb, sem).wait()
        pltpu.make_async_copy(wy_hbm.at[pl.ds(base, EB)], yb, sem).wait()
        pltpu.make_async_copy(wz_hbm.at[pl.ds(base, EB)], zb, sem).wait()
        pltpu.sync_copy(mb, sh_agg.at[idx_d.at[j]], add=True)
        pltpu.sync_copy(xb, sh_dpf.at[fix], add=True)
        pltpu.sync_copy(yb, sh_dpf.at[fiy], add=True)
        pltpu.sync_copy(zb, sh_dpf.at[fiz], add=True)

    start(0, 0)

    def body(k, carry):
        j0 = 2 * k
        j1 = j0 + 1
        start(j1, 1)
        process(j0, 0)

        @pl.when(k < NB // 2 - 1)
        def _():
            start(j0 + 2, 0)

        process(j1, 1)
        return carry

    lax.fori_loop(0, NB // 2, body, 0)
    plsc.subcore_barrier()
    pltpu.sync_copy(sh_agg.at[pl.ds(r0, ROWS_PER_TILE)],
                    agg_out.at[cid, pl.ds(r0, ROWS_PER_TILE)])
    pltpu.sync_copy(sh_dpf.at[pl.ds(f0, DPF_PER_TILE)],
                    dpf_out.at[cid, pl.ds(f0, DPF_PER_TILE)])


def _run_scatter(m, wx, wy, wz, dstm, zagg, zdp):
    f32 = jnp.float32
    return pl.kernel(
        _scatter_body,
        out_type=(
            jax.ShapeDtypeStruct((NC, NP_PAD, HID), f32),
            jax.ShapeDtypeStruct((NC, DPF), f32),
        ),
        mesh=_mesh(),
        compiler_params=pltpu.CompilerParams(needs_layout_passes=False),
        scratch_types=[
            pltpu.VMEM((NB, EB), jnp.int32),
            pltpu.VMEM((EB, HID), f32),
            pltpu.VMEM((EB,), f32),
            pltpu.VMEM((EB,), f32),
            pltpu.VMEM((EB,), f32),
            pltpu.VMEM((EB, HID), f32),
            pltpu.VMEM((EB,), f32),
            pltpu.VMEM((EB,), f32),
            pltpu.VMEM((EB,), f32),
            pltpu.VMEM((EB,), jnp.int32),
            pltpu.VMEM((EB,), jnp.int32),
            pltpu.VMEM((EB,), jnp.int32),
            pltpu.VMEM_SHARED((NP_PAD, HID), f32),
            pltpu.VMEM_SHARED((DPF,), f32),
            pltpu.SemaphoreType.DMA,
            pltpu.SemaphoreType.DMA,
        ],
    )(m, wx, wy, wz, dstm, zagg, zdp)


def _node_body(h, pos, agg2, dp2, w1h, w1g, b1n, w2n, b2n, ew1a, ew1b, eb1,
               h_out, pos_out, ta_out, tb_out):
    f32 = jnp.float32
    hv = h[...]
    a2 = agg2[...]
    agg = a2[0, :N_TOT] + a2[1, :N_TOT]
    u = jnp.maximum(jnp.dot(hv, w1h[...]) + jnp.dot(agg, w1g[...]) + b1n[...],
                    0.0)
    hn = hv + jnp.dot(u, w2n[...]) + b2n[...]
    d2v = dp2[...]
    dp = d2v[0] + d2v[1]
    mask = (lax.broadcasted_iota(jnp.int32, (N_TOT, 1), 0)
            >= N_PROT).astype(f32)
    posn = pos[...] + dp * mask
    h_out[...] = hn
    pos_out[...] = posn
    ta_out[...] = jnp.dot(hn, ew1a[...]) + eb1[...]
    tb_out[...] = jnp.dot(hn, ew1b[...])


def _run_node(h, pos, agg2, dp2, w1h, w1g, b1n, w2n, b2n, ew1a, ew1b, eb1):
    f32 = jnp.float32
    return pl.pallas_call(
        _node_body,
        out_shape=(
            jax.ShapeDtypeStruct((N_TOT, HID), f32),
            jax.ShapeDtypeStruct((N_TOT, 4), f32),
            jax.ShapeDtypeStruct((N_TOT, HID), f32),
            jax.ShapeDtypeStruct((N_TOT, HID), f32),
        ),
    )(h, pos, agg2, dp2, w1h, w1g, b1n, w2n, b2n, ew1a, ew1b, eb1)


def _final_body(hl, posl, offl, vw, vb, pos_out, v_out):
    pos_out[...] = posl[...] + offl[...]
    v_out[...] = jnp.dot(hl[...], vw[...]) + vb[...]


def _run_final(hl, posl, offl, vw, vb):
    f32 = jnp.float32
    return pl.pallas_call(
        _final_body,
        out_shape=(
            jax.ShapeDtypeStruct((N_LIG, 4), f32),
            jax.ShapeDtypeStruct((N_LIG, NUM_CLASSES), f32),
        ),
    )(hl, posl, offl, vw, vb)


def _pack_pos(pos):
    return jnp.reshape(jnp.pad(pos, ((0, NP_PAD - N_TOT), (0, 0))),
                       (NP_PAD * 4,))


def kernel(protein_pos, protein_v, batch_protein, init_ligand_pos,
           init_ligand_v, batch_ligand, time_step, edge_index,
           W_protein, b_protein, W_ligand, b_ligand,
           edge_w1, edge_b1, edge_w2, edge_b2,
           node_w1, node_b1, node_w2, node_b2,
           coord_w, coord_b, v_out_w, v_out_b):
    f32 = jnp.float32
    i32 = jnp.int32

    ppos4 = jnp.pad(protein_pos.astype(f32), ((0, 0), (0, 1)))
    lpos4 = jnp.pad(init_ligand_pos.astype(f32), ((0, 0), (0, 1)))
    bp_col = batch_protein.astype(i32).reshape(N_PROT, 1)
    bp_row = batch_protein.astype(i32).reshape(1, N_PROT)
    bl_col = batch_ligand.astype(i32).reshape(N_LIG, 1)
    bl_row = batch_ligand.astype(i32).reshape(1, N_LIG)
    t_col = time_step.astype(f32).reshape(NUM_GRAPHS, 1)

    wp = jnp.pad(W_protein, ((0, 0), (0, 1)))
    bpb = jnp.pad(b_protein, (0, 1)).reshape(1, HID)
    wlv = jnp.pad(W_ligand[:NUM_CLASSES], ((0, 0), (0, 1)))
    wlt = jnp.pad(W_ligand[NUM_CLASSES:], ((0, 0), (0, 1)))
    blb = jnp.concatenate([b_ligand, jnp.ones((1,), f32)]).reshape(1, HID)
    w1a = edge_w1[:HID]
    w1b = edge_w1[HID:2 * HID]
    w1c = edge_w1[2 * HID:]
    b1 = edge_b1.reshape(1, HID)
    b2 = edge_b2.reshape(1, HID)
    cb = coord_b.reshape(1, 1)
    w1h = node_w1[:HID]
    w1g = node_w1[HID:]
    b1n = node_b1.reshape(1, HID)
    b2n = node_b2.reshape(1, HID)
    vb = v_out_b.reshape(1, NUM_CLASSES)

    src = edge_index[0].astype(i32)
    dst = edge_index[1].astype(i32)
    pad_idx = (jnp.arange(E_PAD - N_EDGE, dtype=i32)) % N_TOT
    srcm = jnp.concatenate([src, pad_idx]).reshape(NW * NB, EB)
    dstm = jnp.concatenate([dst, pad_idx]).reshape(NW * NB, EB)

    zagg = jnp.zeros((NP_PAD, HID), f32)
    zdp = jnp.zeros((DPF,), f32)

    h, pos, offl, ta, tb = _run_pre(
        ppos4, protein_v, bp_col, bp_row, lpos4, init_ligand_v, bl_col,
        bl_row, t_col, wp, bpb, wlv, wlt, blb, w1a, w1b, b1)

    for _ in range(NUM_LAYERS):
        posp = _pack_pos(pos)
        asum, rx, ry, rz = _run_gather(ta, tb, posp, srcm, dstm)
        m, wx, wy, wz = _run_edge(asum, rx, ry, rz, w1c, edge_w2, b2,
                                  coord_w, cb)
        agg2, dpf2 = _run_scatter(m, wx, wy, wz, dstm, zagg, zdp)
        dp2 = dpf2.reshape(NC, NP_PAD, 4)[:, :N_TOT]
        h, pos, ta, tb = _run_node(
            h, pos, agg2, dp2, w1h, w1g, b1n, node_w2, b2n, w1a, w1b, b1)

    posl4, pred_v = _run_final(
        h[N_PROT:], pos[N_PROT:], offl, v_out_w, vb)
    return posl4[:, :3], pred_v


# speedup vs baseline: 5.8598x; 1.1555x over previous
import functools
import math

import jax
import jax.numpy as jnp
from jax import lax
from jax.experimental import pallas as pl
from jax.experimental.pallas import tpu as pltpu
from jax.experimental.pallas import tpu_sc as plsc

N_PROT = 8000
N_LIG = 2000
N_TOT = 10000
N_EDGE = 320000
NUM_GRAPHS = 16
HID = 128
NUM_RBF = 16
R_MAX = 10.0
NUM_LAYERS = 2
NUM_CLASSES = 13

NC = 2
NS = 16
NW = NC * NS
EB = 128
NB = 80
EW = NB * EB
E_PAD = NW * EW
NP_PAD = 10240
DPF = NP_PAD * 4
ROWS_PER_TILE = NP_PAD // NS
DPF_PER_TILE = DPF // NS


@functools.cache
def _mesh():
    return plsc.VectorSubcoreMesh(
        core_axis_name="c", subcore_axis_name="s",
        num_cores=NC, num_subcores=NS)


def _pre_body(ppos, pv, bp_col, bp_row, lpos, lv, bl_col, bl_row, t_col,
              wp, bpb, wlv, wlt, blb, w1a, w1b, b1,
              h_out, pos_out, offl_out, ta_out, tb_out):
    f32 = jnp.float32
    ppos_v = ppos[...]
    lpos_v = lpos[...]
    ohp_t = (lax.broadcasted_iota(jnp.int32, (NUM_GRAPHS, N_PROT), 0)
             == bp_row[...]).astype(f32)
    sums = jnp.dot(ohp_t, ppos_v)
    cnt = jnp.sum(ohp_t, axis=1, keepdims=True)
    mean = sums / jnp.maximum(cnt, 1.0)
    ohp = (lax.broadcasted_iota(jnp.int32, (N_PROT, NUM_GRAPHS), 1)
           == bp_col[...]).astype(f32)
    ohl = (lax.broadcasted_iota(jnp.int32, (N_LIG, NUM_GRAPHS), 1)
           == bl_col[...]).astype(f32)
    p_c = ppos_v - jnp.dot(ohp, mean)
    offl = jnp.dot(ohl, mean)
    l_c = lpos_v - offl
    half = 4
    freq = jnp.exp(lax.broadcasted_iota(jnp.int32, (1, half), 1).astype(f32)
                   * (-math.log(10000.0) / (half - 1)))
    ang = t_col[...] * freq
    te = jnp.concatenate([jnp.sin(ang), jnp.cos(ang)], axis=1)
    te_l = jnp.dot(ohl, te)
    h_p = jnp.dot(pv[...], wp[...]) + bpb[...]
    h_l = jnp.dot(lv[...], wlv[...]) + jnp.dot(te_l, wlt[...]) + blb[...]
    h = jnp.concatenate([h_p, h_l], axis=0)
    pos = jnp.concatenate([p_c, l_c], axis=0)
    h_out[...] = h
    pos_out[...] = pos
    offl_out[...] = offl
    zrows = jnp.zeros((NP_PAD - N_TOT, HID), f32)
    ta_out[...] = jnp.concatenate([jnp.dot(h, w1a[...]) + b1[...], zrows],
                                  axis=0)
    tb_out[...] = jnp.concatenate([jnp.dot(h, w1b[...]), zrows], axis=0)


def _run_pre(ppos4, pv, bp_col, bp_row, lpos4, lv, bl_col, bl_row, t_col,
             wp, bpb, wlv, wlt, blb, w1a, w1b, b1):
    f32 = jnp.float32
    return pl.pallas_call(
        _pre_body,
        out_shape=(
            jax.ShapeDtypeStruct((N_TOT, HID), f32),
            jax.ShapeDtypeStruct((N_TOT, 4), f32),
            jax.ShapeDtypeStruct((N_LIG, 4), f32),
            jax.ShapeDtypeStruct((NP_PAD, HID), f32),
            jax.ShapeDtypeStruct((NP_PAD, HID), f32),
        ),
    )(ppos4, pv, bp_col, bp_row, lpos4, lv, bl_col, bl_row, t_col,
      wp, bpb, wlv, wlt, blb, w1a, w1b, b1)


def _gather_body(ta_hbm, tb_hbm, posp_hbm, srcm_hbm, dstm_hbm,
                 as_out, rx_out, ry_out, rz_out,
                 idx_s, idx_d, posv,
                 ra0, rb0, rx0, ry0, rz0,
                 ra1, rb1, rx1, ry1, rz1,
                 gsem0, gsem1, osem0, osem1):
    cid = lax.axis_index("c")
    sid = lax.axis_index("s")
    wid = cid * NS + sid
    pltpu.sync_copy(srcm_hbm.at[pl.ds(wid * NB, NB)], idx_s)
    pltpu.sync_copy(dstm_hbm.at[pl.ds(wid * NB, NB)], idx_d)
    pltpu.sync_copy(posp_hbm, posv)

    sets = ((ra0, rb0, rx0, ry0, rz0, gsem0, osem0),
            (ra1, rb1, rx1, ry1, rz1, gsem1, osem1))

    def start(j, p):
        ra, rb = sets[p][0], sets[p][1]
        gsem = sets[p][5]
        pltpu.async_copy(ta_hbm.at[idx_s.at[j]], ra, gsem)
        pltpu.async_copy(tb_hbm.at[idx_d.at[j]], rb, gsem)

    def drain_out(j, p):
        ra, _, rxb, ryb, rzb, _, osem = sets[p]
        base = wid * EW + j * EB
        pltpu.make_async_copy(ra, as_out.at[pl.ds(base, EB)], osem).wait()
        pltpu.make_async_copy(rxb, rx_out.at[pl.ds(base, EB)], osem).wait()
        pltpu.make_async_copy(ryb, ry_out.at[pl.ds(base, EB)], osem).wait()
        pltpu.make_async_copy(rzb, rz_out.at[pl.ds(base, EB)], osem).wait()

    def process(j, p):
        ra, rb, rxb, ryb, rzb, gsem, osem = sets[p]

        def grp(gi, c2):
            s16 = idx_s[j, pl.ds(gi * 16, 16)]
            d16 = idx_d[j, pl.ds(gi * 16, 16)]
            fs = lax.shift_left(s16, 2)
            fd = lax.shift_left(d16, 2)
            for c, buf in ((0, rxb), (1, ryb), (2, rzb)):
                ps = plsc.load_gather(posv, [fs + c])
                pd = plsc.load_gather(posv, [fd + c])
                buf[pl.ds(gi * 16, 16)] = pd - ps
            return c2

        lax.fori_loop(0, EB // 16, grp, 0)
        pltpu.make_async_copy(ta_hbm.at[idx_s.at[j]], ra, gsem).wait()
        pltpu.make_async_copy(tb_hbm.at[idx_d.at[j]], rb, gsem).wait()

        @plsc.parallel_loop(0, EB * HID // 16, unroll=8)
        def _add(i):
            r = lax.shift_right_logical(i, 3)
            g = lax.bitwise_and(i, 7)
            ra[r, pl.ds(g * 16, 16)] += rb[r, pl.ds(g * 16, 16)]

        base = wid * EW + j * EB
        pltpu.async_copy(ra, as_out.at[pl.ds(base, EB)], osem)
        pltpu.async_copy(rxb, rx_out.at[pl.ds(base, EB)], osem)
        pltpu.async_copy(ryb, ry_out.at[pl.ds(base, EB)], osem)
        pltpu.async_copy(rzb, rz_out.at[pl.ds(base, EB)], osem)

    start(0, 0)

    def body(k, carry):
        j0 = 2 * k
        j1 = j0 + 1
        process(j0, 0)

        @pl.when(k > 0)
        def _():
            drain_out(j1 - 2, 1)

        start(j1, 1)
        process(j1, 1)
        drain_out(j0, 0)

        @pl.when(k < NB // 2 - 1)
        def _():
            start(j0 + 2, 0)

        return carry

    lax.fori_loop(0, NB // 2, body, 0)
    drain_out(NB - 1, 1)


def _run_gather(ta, tb, posp, srcm, dstm):
    f32 = jnp.float32
    return pl.kernel(
        _gather_body,
        out_type=(
            jax.ShapeDtypeStruct((E_PAD, HID), f32),
            jax.ShapeDtypeStruct((E_PAD,), f32),
            jax.ShapeDtypeStruct((E_PAD,), f32),
            jax.ShapeDtypeStruct((E_PAD,), f32),
        ),
        mesh=_mesh(),
        compiler_params=pltpu.CompilerParams(needs_layout_passes=False),
        scratch_types=[
            pltpu.VMEM((NB, EB), jnp.int32),
            pltpu.VMEM((NB, EB), jnp.int32),
            pltpu.VMEM((NP_PAD * 4,), f32),
            pltpu.VMEM((EB, HID), f32),
            pltpu.VMEM((EB, HID), f32),
            pltpu.VMEM((EB,), f32),
            pltpu.VMEM((EB,), f32),
            pltpu.VMEM((EB,), f32),
            pltpu.VMEM((EB, HID), f32),
            pltpu.VMEM((EB, HID), f32),
            pltpu.VMEM((EB,), f32),
            pltpu.VMEM((EB,), f32),
            pltpu.VMEM((EB,), f32),
            pltpu.SemaphoreType.DMA,
            pltpu.SemaphoreType.DMA,
            pltpu.SemaphoreType.DMA,
            pltpu.SemaphoreType.DMA,
        ],
    )(ta, tb, posp, srcm, dstm)


_EBLK = 2048


def _edge_body(asum, rx, ry, rz, w1c, w2, b2, cw, cb,
               m_out, wx_out, wy_out, wz_out):
    f32 = jnp.float32
    rxv = rx[...]
    ryv = ry[...]
    rzv = rz[...]
    d_flat = jnp.sqrt(rxv * rxv + ryv * ryv + rzv * rzv)
    d_row = d_flat.reshape(1, _EBLK)
    delta = R_MAX / (NUM_RBF - 1)
    cent_col = (lax.broadcasted_iota(jnp.int32, (NUM_RBF, 1), 0)
                .astype(f32) * delta)
    rbf_t = jnp.exp((-0.5 / (delta * delta)) * (d_row - cent_col) ** 2)
    contrib = lax.dot_general(rbf_t, w1c[...],
                              (((0,), (0,)), ((), ())))
    m1 = jnp.maximum(asum[...] + contrib, 0.0)
    m = jnp.maximum(jnp.dot(m1, w2[...]) + b2[...], 0.0)
    cw_row = lax.dot_general(cw[...], m,
                             (((0,), (1,)), ((), ())))
    w_row = (cw_row + cb[...]) / (d_row + 1.0)
    w_flat = jnp.squeeze(w_row, axis=0)
    m_out[...] = m
    wx_out[...] = rxv * w_flat
    wy_out[...] = ryv * w_flat
    wz_out[...] = rzv * w_flat


def _run_edge(asum, rx, ry, rz, w1c, w2, b2, cw, cb):
    f32 = jnp.float32
    nblk = E_PAD // _EBLK
    full = lambda i: (0, 0)
    return pl.pallas_call(
        _edge_body,
        grid=(nblk,),
        in_specs=[
            pl.BlockSpec((_EBLK, HID), lambda i: (i, 0)),
            pl.BlockSpec((_EBLK,), lambda i: (i,)),
            pl.BlockSpec((_EBLK,), lambda i: (i,)),
            pl.BlockSpec((_EBLK,), lambda i: (i,)),
            pl.BlockSpec((NUM_RBF, HID), full),
            pl.BlockSpec((HID, HID), full),
            pl.BlockSpec((1, HID), full),
            pl.BlockSpec((HID, 1), full),
            pl.BlockSpec((1, 1), full),
        ],
        out_specs=[
            pl.BlockSpec((_EBLK, HID), lambda i: (i, 0)),
            pl.BlockSpec((_EBLK,), lambda i: (i,)),
            pl.BlockSpec((_EBLK,), lambda i: (i,)),
            pl.BlockSpec((_EBLK,), lambda i: (i,)),
        ],
        out_shape=(
            jax.ShapeDtypeStruct((E_PAD, HID), f32),
            jax.ShapeDtypeStruct((E_PAD,), f32),
            jax.ShapeDtypeStruct((E_PAD,), f32),
            jax.ShapeDtypeStruct((E_PAD,), f32),
        ),
    )(asum, rx, ry, rz, w1c, w2, b2, cw, cb)


def _scatter_body(m_hbm, wx_hbm, wy_hbm, wz_hbm, dstm_hbm, zagg_hbm, zdp_hbm,
                  agg_out, dpf_out,
                  idx_d, mr0, xb0, yb0, zb0, mr1, xb1, yb1, zb1,
                  fix, fiy, fiz, sh_agg, sh_dpf, sem0, sem1):
    cid = lax.axis_index("c")
    sid = lax.axis_index("s")
    wid = cid * NS + sid
    r0 = sid * ROWS_PER_TILE
    f0 = sid * DPF_PER_TILE
    pltpu.sync_copy(zagg_hbm.at[pl.ds(r0, ROWS_PER_TILE)],
                    sh_agg.at[pl.ds(r0, ROWS_PER_TILE)])
    pltpu.sync_copy(zdp_hbm.at[pl.ds(f0, DPF_PER_TILE)],
                    sh_dpf.at[pl.ds(f0, DPF_PER_TILE)])
    pltpu.sync_copy(dstm_hbm.at[pl.ds(wid * NB, NB)], idx_d)
    plsc.subcore_barrier()

    sets = ((mr0, xb0, yb0, zb0, sem0), (mr1, xb1, yb1, zb1, sem1))

    def start(j, p):
        mb, xb, yb, zb, sem = sets[p]
        base = wid * EW + j * EB
        pltpu.async_copy(m_hbm.at[pl.ds(base, EB)], mb, sem)
        pltpu.async_copy(wx_hbm.at[pl.ds(base, EB)], xb, sem)
        pltpu.async_copy(wy_hbm.at[pl.ds(base, EB)], yb, sem)
        pltpu.async_copy(wz_hbm.at[pl.ds(base, EB)], zb, sem)

    def process(j, p):
        mb, xb, yb, zb, sem = sets[p]

        def grp(gi, c2):
            d16 = idx_d[j, pl.ds(gi * 16, 16)]
            f = lax.shift_left(d16, 2)
            fix[pl.ds(gi * 16, 16)] = f
            fiy[pl.ds(gi * 16, 16)] = f + 1
            fiz[pl.ds(gi * 16, 16)] = f + 2
            return c2

        lax.fori_loop(0, EB // 16, grp, 0)
        base = wid * EW + j * EB
        pltpu.make_async_copy(m_hbm.at[pl.ds(base, EB)], mb, sem).wait()
        pltpu.make_async_copy(wx_hbm.at[pl.ds(base, EB)], xb, sem).wait()
        pltpu.make_async_copy(wy_hbm.at[pl.ds(base, EB)], yb, sem).wait()
        pltpu.make_async_copy(wz_hbm.at[pl.ds(base, EB)], zb, sem).wait()
        pltpu.sync_copy(mb, sh_agg.at[idx_d.at[j]], add=True)
        pltpu.sync_copy(xb, sh_dpf.at[fix], add=True)
        pltpu.sync_copy(yb, sh_dpf.at[fiy], add=True)
        pltpu.sync_copy(zb, sh_dpf.at[fiz], add=True)

    start(0, 0)

    def body(k, carry):
        j0 = 2 * k
        j1 = j0 + 1
        start(j1, 1)
        process(j0, 0)

        @pl.when(k < NB // 2 - 1)
        def _():
            start(j0 + 2, 0)

        process(j1, 1)
        return carry

    lax.fori_loop(0, NB // 2, body, 0)
    plsc.subcore_barrier()
    pltpu.sync_copy(sh_agg.at[pl.ds(r0, ROWS_PER_TILE)],
                    agg_out.at[cid, pl.ds(r0, ROWS_PER_TILE)])
    pltpu.sync_copy(sh_dpf.at[pl.ds(f0, DPF_PER_TILE)],
                    dpf_out.at[cid, pl.ds(f0, DPF_PER_TILE)])


def _run_scatter(m, wx, wy, wz, dstm, zagg, zdp):
    f32 = jnp.float32
    return pl.kernel(
        _scatter_body,
        out_type=(
            jax.ShapeDtypeStruct((NC, NP_PAD, HID), f32),
            jax.ShapeDtypeStruct((NC, DPF), f32),
        ),
        mesh=_mesh(),
        compiler_params=pltpu.CompilerParams(needs_layout_passes=False),
        scratch_types=[
            pltpu.VMEM((NB, EB), jnp.int32),
            pltpu.VMEM((EB, HID), f32),
            pltpu.VMEM((EB,), f32),
            pltpu.VMEM((EB,), f32),
            pltpu.VMEM((EB,), f32),
            pltpu.VMEM((EB, HID), f32),
            pltpu.VMEM((EB,), f32),
            pltpu.VMEM((EB,), f32),
            pltpu.VMEM((EB,), f32),
            pltpu.VMEM((EB,), jnp.int32),
            pltpu.VMEM((EB,), jnp.int32),
            pltpu.VMEM((EB,), jnp.int32),
            pltpu.VMEM_SHARED((NP_PAD, HID), f32),
            pltpu.VMEM_SHARED((DPF,), f32),
            pltpu.SemaphoreType.DMA,
            pltpu.SemaphoreType.DMA,
        ],
    )(m, wx, wy, wz, dstm, zagg, zdp)


def _node_body(h, pos, agg2, dp2, w1h, w1g, b1n, w2n, b2n, ew1a, ew1b, eb1,
               h_out, pos_out, ta_out, tb_out):
    f32 = jnp.float32
    hv = h[...]
    a2 = agg2[...]
    agg = a2[0, :N_TOT] + a2[1, :N_TOT]
    u = jnp.maximum(jnp.dot(hv, w1h[...]) + jnp.dot(agg, w1g[...]) + b1n[...],
                    0.0)
    hn = hv + jnp.dot(u, w2n[...]) + b2n[...]
    d2v = dp2[...]
    dp = d2v[0] + d2v[1]
    mask = (lax.broadcasted_iota(jnp.int32, (N_TOT, 1), 0)
            >= N_PROT).astype(f32)
    posn = pos[...] + dp * mask
    h_out[...] = hn
    pos_out[...] = posn
    zrows = jnp.zeros((NP_PAD - N_TOT, HID), f32)
    ta_out[...] = jnp.concatenate([jnp.dot(hn, ew1a[...]) + eb1[...], zrows],
                                  axis=0)
    tb_out[...] = jnp.concatenate([jnp.dot(hn, ew1b[...]), zrows], axis=0)


def _run_node(h, pos, agg2, dp2, w1h, w1g, b1n, w2n, b2n, ew1a, ew1b, eb1):
    f32 = jnp.float32
    return pl.pallas_call(
        _node_body,
        out_shape=(
            jax.ShapeDtypeStruct((N_TOT, HID), f32),
            jax.ShapeDtypeStruct((N_TOT, 4), f32),
            jax.ShapeDtypeStruct((NP_PAD, HID), f32),
            jax.ShapeDtypeStruct((NP_PAD, HID), f32),
        ),
    )(h, pos, agg2, dp2, w1h, w1g, b1n, w2n, b2n, ew1a, ew1b, eb1)


def _final_body(hl, posl, offl, vw, vb, pos_out, v_out):
    pos_out[...] = posl[...] + offl[...]
    v_out[...] = jnp.dot(hl[...], vw[...]) + vb[...]


def _run_final(hl, posl, offl, vw, vb):
    f32 = jnp.float32
    return pl.pallas_call(
        _final_body,
        out_shape=(
            jax.ShapeDtypeStruct((N_LIG, 4), f32),
            jax.ShapeDtypeStruct((N_LIG, NUM_CLASSES), f32),
        ),
    )(hl, posl, offl, vw, vb)


def _pack_pos(pos):
    return jnp.reshape(jnp.pad(pos, ((0, NP_PAD - N_TOT), (0, 0))),
                       (NP_PAD * 4,))


def kernel(protein_pos, protein_v, batch_protein, init_ligand_pos,
           init_ligand_v, batch_ligand, time_step, edge_index,
           W_protein, b_protein, W_ligand, b_ligand,
           edge_w1, edge_b1, edge_w2, edge_b2,
           node_w1, node_b1, node_w2, node_b2,
           coord_w, coord_b, v_out_w, v_out_b):
    f32 = jnp.float32
    i32 = jnp.int32

    ppos4 = jnp.pad(protein_pos.astype(f32), ((0, 0), (0, 1)))
    lpos4 = jnp.pad(init_ligand_pos.astype(f32), ((0, 0), (0, 1)))
    bp_col = batch_protein.astype(i32).reshape(N_PROT, 1)
    bp_row = batch_protein.astype(i32).reshape(1, N_PROT)
    bl_col = batch_ligand.astype(i32).reshape(N_LIG, 1)
    bl_row = batch_ligand.astype(i32).reshape(1, N_LIG)
    t_col = time_step.astype(f32).reshape(NUM_GRAPHS, 1)

    wp = jnp.pad(W_protein, ((0, 0), (0, 1)))
    bpb = jnp.pad(b_protein, (0, 1)).reshape(1, HID)
    wlv = jnp.pad(W_ligand[:NUM_CLASSES], ((0, 0), (0, 1)))
    wlt = jnp.pad(W_ligand[NUM_CLASSES:], ((0, 0), (0, 1)))
    blb = jnp.concatenate([b_ligand, jnp.ones((1,), f32)]).reshape(1, HID)
    w1a = edge_w1[:HID]
    w1b = edge_w1[HID:2 * HID]
    w1c = edge_w1[2 * HID:]
    b1 = edge_b1.reshape(1, HID)
    b2 = edge_b2.reshape(1, HID)
    cb = coord_b.reshape(1, 1)
    w1h = node_w1[:HID]
    w1g = node_w1[HID:]
    b1n = node_b1.reshape(1, HID)
    b2n = node_b2.reshape(1, HID)
    vb = v_out_b.reshape(1, NUM_CLASSES)

    src = edge_index[0].astype(i32)
    dst = edge_index[1].astype(i32)
    pad_idx = N_TOT + jnp.arange(E_PAD - N_EDGE, dtype=i32) % (NP_PAD - N_TOT)
    srcm = jnp.concatenate([src, pad_idx]).reshape(NW * NB, EB)
    dstm = jnp.concatenate([dst, pad_idx]).reshape(NW * NB, EB)

    zagg = jnp.zeros((NP_PAD, HID), f32)
    zdp = jnp.zeros((DPF,), f32)

    h, pos, offl, ta, tb = _run_pre(
        ppos4, protein_v, bp_col, bp_row, lpos4, init_ligand_v, bl_col,
        bl_row, t_col, wp, bpb, wlv, wlt, blb, w1a, w1b, b1)

    for _ in range(NUM_LAYERS):
        posp = _pack_pos(pos)
        asum, rx, ry, rz = _run_gather(ta, tb, posp, srcm, dstm)
        m, wx, wy, wz = _run_edge(asum, rx, ry, rz, w1c, edge_w2, b2,
                                  coord_w, cb)
        agg2, dpf2 = _run_scatter(m, wx, wy, wz, dstm, zagg, zdp)
        dp2 = dpf2.reshape(NC, NP_PAD, 4)[:, :N_TOT]
        h, pos, ta, tb = _run_node(
            h, pos, agg2, dp2, w1h, w1g, b1n, node_w2, b2n, w1a, w1b, b1)

    posl4, pred_v = _run_final(
        h[N_PROT:], pos[N_PROT:], offl, v_out_w, vb)
    return posl4[:, :3], pred_v


# speedup vs baseline: 6.4794x; 1.1057x over previous
import functools
import math

import jax
import jax.numpy as jnp
from jax import lax
from jax.experimental import pallas as pl
from jax.experimental.pallas import tpu as pltpu
from jax.experimental.pallas import tpu_sc as plsc

N_PROT = 8000
N_LIG = 2000
N_TOT = 10000
N_EDGE = 320000
NUM_GRAPHS = 16
HID = 128
NUM_RBF = 16
R_MAX = 10.0
NUM_LAYERS = 2
NUM_CLASSES = 13

NC = 2
NS = 16
NW = NC * NS
EB = 128
NB = 40
EW = NB * EB
NCHUNK = 2
E_CH = NW * EW
E_PAD = NCHUNK * E_CH
NP_PAD = 10240
DPF = NP_PAD * 4
ROWS_PER_TILE = NP_PAD // NS
DPF_PER_TILE = DPF // NS


@functools.cache
def _mesh():
    return plsc.VectorSubcoreMesh(
        core_axis_name="c", subcore_axis_name="s",
        num_cores=NC, num_subcores=NS)


def _pre_body(ppos, pv, bp_col, bp_row, lpos, lv, bl_col, bl_row, t_col,
              wp, bpb, wlv, wlt, blb, w1a, w1b, b1,
              h_out, pos_out, offl_out, ta_out, tb_out):
    f32 = jnp.float32
    ppos_v = ppos[...]
    lpos_v = lpos[...]
    ohp_t = (lax.broadcasted_iota(jnp.int32, (NUM_GRAPHS, N_PROT), 0)
             == bp_row[...]).astype(f32)
    sums = jnp.dot(ohp_t, ppos_v)
    cnt = jnp.sum(ohp_t, axis=1, keepdims=True)
    mean = sums / jnp.maximum(cnt, 1.0)
    ohp = (lax.broadcasted_iota(jnp.int32, (N_PROT, NUM_GRAPHS), 1)
           == bp_col[...]).astype(f32)
    ohl = (lax.broadcasted_iota(jnp.int32, (N_LIG, NUM_GRAPHS), 1)
           == bl_col[...]).astype(f32)
    p_c = ppos_v - jnp.dot(ohp, mean)
    offl = jnp.dot(ohl, mean)
    l_c = lpos_v - offl
    half = 4
    freq = jnp.exp(lax.broadcasted_iota(jnp.int32, (1, half), 1).astype(f32)
                   * (-math.log(10000.0) / (half - 1)))
    ang = t_col[...] * freq
    te = jnp.concatenate([jnp.sin(ang), jnp.cos(ang)], axis=1)
    te_l = jnp.dot(ohl, te)
    h_p = jnp.dot(pv[...], wp[...]) + bpb[...]
    h_l = jnp.dot(lv[...], wlv[...]) + jnp.dot(te_l, wlt[...]) + blb[...]
    h = jnp.concatenate([h_p, h_l], axis=0)
    pos = jnp.concatenate([p_c, l_c], axis=0)
    h_out[...] = h
    pos_out[...] = pos
    offl_out[...] = offl
    zrows = jnp.zeros((NP_PAD - N_TOT, HID), f32)
    ta_out[...] = jnp.concatenate([jnp.dot(h, w1a[...]) + b1[...], zrows],
                                  axis=0)
    tb_out[...] = jnp.concatenate([jnp.dot(h, w1b[...]), zrows], axis=0)


def _run_pre(ppos4, pv, bp_col, bp_row, lpos4, lv, bl_col, bl_row, t_col,
             wp, bpb, wlv, wlt, blb, w1a, w1b, b1):
    f32 = jnp.float32
    return pl.pallas_call(
        _pre_body,
        out_shape=(
            jax.ShapeDtypeStruct((N_TOT, HID), f32),
            jax.ShapeDtypeStruct((N_TOT, 4), f32),
            jax.ShapeDtypeStruct((N_LIG, 4), f32),
            jax.ShapeDtypeStruct((NP_PAD, HID), f32),
            jax.ShapeDtypeStruct((NP_PAD, HID), f32),
        ),
    )(ppos4, pv, bp_col, bp_row, lpos4, lv, bl_col, bl_row, t_col,
      wp, bpb, wlv, wlt, blb, w1a, w1b, b1)


def _gather_body(ta_hbm, tb_hbm, posp_hbm, srcm_hbm, dstm_hbm,
                 as_out, rx_out, ry_out, rz_out,
                 idx_s, idx_d, posv,
                 ra0, rb0, rx0, ry0, rz0,
                 ra1, rb1, rx1, ry1, rz1,
                 gsem0, gsem1, osem0, osem1):
    cid = lax.axis_index("c")
    sid = lax.axis_index("s")
    wid = cid * NS + sid
    pltpu.sync_copy(srcm_hbm.at[pl.ds(wid * NB, NB)], idx_s)
    pltpu.sync_copy(dstm_hbm.at[pl.ds(wid * NB, NB)], idx_d)
    pltpu.sync_copy(posp_hbm, posv)

    sets = ((ra0, rb0, rx0, ry0, rz0, gsem0, osem0),
            (ra1, rb1, rx1, ry1, rz1, gsem1, osem1))

    def start(j, p):
        ra, rb = sets[p][0], sets[p][1]
        gsem = sets[p][5]
        pltpu.async_copy(ta_hbm.at[idx_s.at[j]], ra, gsem)
        pltpu.async_copy(tb_hbm.at[idx_d.at[j]], rb, gsem)

    def drain_out(j, p):
        ra, _, rxb, ryb, rzb, _, osem = sets[p]
        base = wid * EW + j * EB
        pltpu.make_async_copy(ra, as_out.at[pl.ds(base, EB)], osem).wait()
        pltpu.make_async_copy(rxb, rx_out.at[pl.ds(base, EB)], osem).wait()
        pltpu.make_async_copy(ryb, ry_out.at[pl.ds(base, EB)], osem).wait()
        pltpu.make_async_copy(rzb, rz_out.at[pl.ds(base, EB)], osem).wait()

    def process(j, p):
        ra, rb, rxb, ryb, rzb, gsem, osem = sets[p]

        def grp(gi, c2):
            s16 = idx_s[j, pl.ds(gi * 16, 16)]
            d16 = idx_d[j, pl.ds(gi * 16, 16)]
            fs = lax.shift_left(s16, 2)
            fd = lax.shift_left(d16, 2)
            for c, buf in ((0, rxb), (1, ryb), (2, rzb)):
                ps = plsc.load_gather(posv, [fs + c])
                pd = plsc.load_gather(posv, [fd + c])
                buf[pl.ds(gi * 16, 16)] = pd - ps
            return c2

        lax.fori_loop(0, EB // 16, grp, 0)
        pltpu.make_async_copy(ta_hbm.at[idx_s.at[j]], ra, gsem).wait()
        pltpu.make_async_copy(tb_hbm.at[idx_d.at[j]], rb, gsem).wait()

        @plsc.parallel_loop(0, EB * HID // 16, unroll=8)
        def _add(i):
            r = lax.shift_right_logical(i, 3)
            g = lax.bitwise_and(i, 7)
            ra[r, pl.ds(g * 16, 16)] += rb[r, pl.ds(g * 16, 16)]

        base = wid * EW + j * EB
        pltpu.async_copy(ra, as_out.at[pl.ds(base, EB)], osem)
        pltpu.async_copy(rxb, rx_out.at[pl.ds(base, EB)], osem)
        pltpu.async_copy(ryb, ry_out.at[pl.ds(base, EB)], osem)
        pltpu.async_copy(rzb, rz_out.at[pl.ds(base, EB)], osem)

    start(0, 0)

    def body(k, carry):
        j0 = 2 * k
        j1 = j0 + 1
        process(j0, 0)

        @pl.when(k > 0)
        def _():
            drain_out(j1 - 2, 1)

        start(j1, 1)
        process(j1, 1)
        drain_out(j0, 0)

        @pl.when(k < NB // 2 - 1)
        def _():
            start(j0 + 2, 0)

        return carry

    lax.fori_loop(0, NB // 2, body, 0)
    drain_out(NB - 1, 1)


def _run_gather(ta, tb, posp, srcm, dstm):
    f32 = jnp.float32
    return pl.kernel(
        _gather_body,
        out_type=(
            jax.ShapeDtypeStruct((E_CH, HID), f32),
            jax.ShapeDtypeStruct((E_CH,), f32),
            jax.ShapeDtypeStruct((E_CH,), f32),
            jax.ShapeDtypeStruct((E_CH,), f32),
        ),
        mesh=_mesh(),
        compiler_params=pltpu.CompilerParams(needs_layout_passes=False),
        scratch_types=[
            pltpu.VMEM((NB, EB), jnp.int32),
            pltpu.VMEM((NB, EB), jnp.int32),
            pltpu.VMEM((NP_PAD * 4,), f32),
            pltpu.VMEM((EB, HID), f32),
            pltpu.VMEM((EB, HID), f32),
            pltpu.VMEM((EB,), f32),
            pltpu.VMEM((EB,), f32),
            pltpu.VMEM((EB,), f32),
            pltpu.VMEM((EB, HID), f32),
            pltpu.VMEM((EB, HID), f32),
            pltpu.VMEM((EB,), f32),
            pltpu.VMEM((EB,), f32),
            pltpu.VMEM((EB,), f32),
            pltpu.SemaphoreType.DMA,
            pltpu.SemaphoreType.DMA,
            pltpu.SemaphoreType.DMA,
            pltpu.SemaphoreType.DMA,
        ],
    )(ta, tb, posp, srcm, dstm)


_EBLK = 2048


def _edge_body(asum, rx, ry, rz, w1c, w2, b2, cw, cb,
               m_out, wx_out, wy_out, wz_out):
    f32 = jnp.float32
    rxv = rx[...]
    ryv = ry[...]
    rzv = rz[...]
    d_flat = jnp.sqrt(rxv * rxv + ryv * ryv + rzv * rzv)
    d_row = d_flat.reshape(1, _EBLK)
    delta = R_MAX / (NUM_RBF - 1)
    cent_col = (lax.broadcasted_iota(jnp.int32, (NUM_RBF, 1), 0)
                .astype(f32) * delta)
    rbf_t = jnp.exp((-0.5 / (delta * delta)) * (d_row - cent_col) ** 2)
    contrib = lax.dot_general(rbf_t, w1c[...],
                              (((0,), (0,)), ((), ())))
    m1 = jnp.maximum(asum[...] + contrib, 0.0)
    m = jnp.maximum(jnp.dot(m1, w2[...]) + b2[...], 0.0)
    cw_row = lax.dot_general(cw[...], m,
                             (((0,), (1,)), ((), ())))
    w_row = (cw_row + cb[...]) / (d_row + 1.0)
    w_flat = jnp.squeeze(w_row, axis=0)
    m_out[...] = m
    wx_out[...] = rxv * w_flat
    wy_out[...] = ryv * w_flat
    wz_out[...] = rzv * w_flat


def _run_edge(asum, rx, ry, rz, w1c, w2, b2, cw, cb):
    f32 = jnp.float32
    nblk = E_CH // _EBLK
    full = lambda i: (0, 0)
    return pl.pallas_call(
        _edge_body,
        grid=(nblk,),
        in_specs=[
            pl.BlockSpec((_EBLK, HID), lambda i: (i, 0)),
            pl.BlockSpec((_EBLK,), lambda i: (i,)),
            pl.BlockSpec((_EBLK,), lambda i: (i,)),
            pl.BlockSpec((_EBLK,), lambda i: (i,)),
            pl.BlockSpec((NUM_RBF, HID), full),
            pl.BlockSpec((HID, HID), full),
            pl.BlockSpec((1, HID), full),
            pl.BlockSpec((HID, 1), full),
            pl.BlockSpec((1, 1), full),
        ],
        out_specs=[
            pl.BlockSpec((_EBLK, HID), lambda i: (i, 0)),
            pl.BlockSpec((_EBLK,), lambda i: (i,)),
            pl.BlockSpec((_EBLK,), lambda i: (i,)),
            pl.BlockSpec((_EBLK,), lambda i: (i,)),
        ],
        out_shape=(
            jax.ShapeDtypeStruct((E_CH, HID), f32),
            jax.ShapeDtypeStruct((E_CH,), f32),
            jax.ShapeDtypeStruct((E_CH,), f32),
            jax.ShapeDtypeStruct((E_CH,), f32),
        ),
    )(asum, rx, ry, rz, w1c, w2, b2, cw, cb)


def _scatter_body(m0_hbm, m1_hbm, wx0_hbm, wx1_hbm, wy0_hbm, wy1_hbm,
                  wz0_hbm, wz1_hbm, dstm0_hbm, dstm1_hbm, zagg_hbm, zdp_hbm,
                  agg_out, dpf_out,
                  idx_d, mr0, xb0, yb0, zb0, mr1, xb1, yb1, zb1,
                  fix, fiy, fiz, sh_agg, sh_dpf, sem0, sem1):
    cid = lax.axis_index("c")
    sid = lax.axis_index("s")
    wid = cid * NS + sid
    r0 = sid * ROWS_PER_TILE
    f0 = sid * DPF_PER_TILE
    pltpu.sync_copy(zagg_hbm.at[pl.ds(r0, ROWS_PER_TILE)],
                    sh_agg.at[pl.ds(r0, ROWS_PER_TILE)])
    pltpu.sync_copy(zdp_hbm.at[pl.ds(f0, DPF_PER_TILE)],
                    sh_dpf.at[pl.ds(f0, DPF_PER_TILE)])
    plsc.subcore_barrier()

    sets = ((mr0, xb0, yb0, zb0, sem0), (mr1, xb1, yb1, zb1, sem1))

    def one_chunk(m_hbm, wx_hbm, wy_hbm, wz_hbm, dstm_hbm):
        pltpu.sync_copy(dstm_hbm.at[pl.ds(wid * NB, NB)], idx_d)

        def start(j, p):
            mb, xb, yb, zb, sem = sets[p]
            base = wid * EW + j * EB
            pltpu.async_copy(m_hbm.at[pl.ds(base, EB)], mb, sem)
            pltpu.async_copy(wx_hbm.at[pl.ds(base, EB)], xb, sem)
            pltpu.async_copy(wy_hbm.at[pl.ds(base, EB)], yb, sem)
            pltpu.async_copy(wz_hbm.at[pl.ds(base, EB)], zb, sem)

        def process(j, p):
            mb, xb, yb, zb, sem = sets[p]

            def grp(gi, c2):
                d16 = idx_d[j, pl.ds(gi * 16, 16)]
                f = lax.shift_left(d16, 2)
                fix[pl.ds(gi * 16, 16)] = f
                fiy[pl.ds(gi * 16, 16)] = f + 1
                fiz[pl.ds(gi * 16, 16)] = f + 2
                return c2

            lax.fori_loop(0, EB // 16, grp, 0)
            base = wid * EW + j * EB
            pltpu.make_async_copy(m_hbm.at[pl.ds(base, EB)], mb, sem).wait()
            pltpu.make_async_copy(wx_hbm.at[pl.ds(base, EB)], xb, sem).wait()
            pltpu.make_async_copy(wy_hbm.at[pl.ds(base, EB)], yb, sem).wait()
            pltpu.make_async_copy(wz_hbm.at[pl.ds(base, EB)], zb, sem).wait()
            pltpu.sync_copy(mb, sh_agg.at[idx_d.at[j]], add=True)
            pltpu.sync_copy(xb, sh_dpf.at[fix], add=True)
            pltpu.sync_copy(yb, sh_dpf.at[fiy], add=True)
            pltpu.sync_copy(zb, sh_dpf.at[fiz], add=True)

        start(0, 0)

        def body(k, carry):
            j0 = 2 * k
            j1 = j0 + 1
            start(j1, 1)
            process(j0, 0)

            @pl.when(k < NB // 2 - 1)
            def _():
                start(j0 + 2, 0)

            process(j1, 1)
            return carry

        lax.fori_loop(0, NB // 2, body, 0)

    one_chunk(m0_hbm, wx0_hbm, wy0_hbm, wz0_hbm, dstm0_hbm)
    one_chunk(m1_hbm, wx1_hbm, wy1_hbm, wz1_hbm, dstm1_hbm)
    plsc.subcore_barrier()
    pltpu.sync_copy(sh_agg.at[pl.ds(r0, ROWS_PER_TILE)],
                    agg_out.at[cid, pl.ds(r0, ROWS_PER_TILE)])
    pltpu.sync_copy(sh_dpf.at[pl.ds(f0, DPF_PER_TILE)],
                    dpf_out.at[cid, pl.ds(f0, DPF_PER_TILE)])


def _run_scatter(m0, m1, wx0, wx1, wy0, wy1, wz0, wz1, dm0, dm1, zagg, zdp):
    f32 = jnp.float32
    return pl.kernel(
        _scatter_body,
        out_type=(
            jax.ShapeDtypeStruct((NC, NP_PAD, HID), f32),
            jax.ShapeDtypeStruct((NC, DPF), f32),
        ),
        mesh=_mesh(),
        compiler_params=pltpu.CompilerParams(needs_layout_passes=False),
        scratch_types=[
            pltpu.VMEM((NB, EB), jnp.int32),
            pltpu.VMEM((EB, HID), f32),
            pltpu.VMEM((EB,), f32),
            pltpu.VMEM((EB,), f32),
            pltpu.VMEM((EB,), f32),
            pltpu.VMEM((EB, HID), f32),
            pltpu.VMEM((EB,), f32),
            pltpu.VMEM((EB,), f32),
            pltpu.VMEM((EB,), f32),
            pltpu.VMEM((EB,), jnp.int32),
            pltpu.VMEM((EB,), jnp.int32),
            pltpu.VMEM((EB,), jnp.int32),
            pltpu.VMEM_SHARED((NP_PAD, HID), f32),
            pltpu.VMEM_SHARED((DPF,), f32),
            pltpu.SemaphoreType.DMA,
            pltpu.SemaphoreType.DMA,
        ],
    )(m0, m1, wx0, wx1, wy0, wy1, wz0, wz1, dm0, dm1, zagg, zdp)


def _node_body(h, pos, agg2, dp2, w1h, w1g, b1n, w2n, b2n, ew1a, ew1b, eb1,
               h_out, pos_out, ta_out, tb_out):
    f32 = jnp.float32
    hv = h[...]
    a2 = agg2[...]
    agg = a2[0, :N_TOT] + a2[1, :N_TOT]
    u = jnp.maximum(jnp.dot(hv, w1h[...]) + jnp.dot(agg, w1g[...]) + b1n[...],
                    0.0)
    hn = hv + jnp.dot(u, w2n[...]) + b2n[...]
    d2v = dp2[...]
    dp = d2v[0] + d2v[1]
    mask = (lax.broadcasted_iota(jnp.int32, (N_TOT, 1), 0)
            >= N_PROT).astype(f32)
    posn = pos[...] + dp * mask
    h_out[...] = hn
    pos_out[...] = posn
    zrows = jnp.zeros((NP_PAD - N_TOT, HID), f32)
    ta_out[...] = jnp.concatenate([jnp.dot(hn, ew1a[...]) + eb1[...], zrows],
                                  axis=0)
    tb_out[...] = jnp.concatenate([jnp.dot(hn, ew1b[...]), zrows], axis=0)


def _run_node(h, pos, agg2, dp2, w1h, w1g, b1n, w2n, b2n, ew1a, ew1b, eb1):
    f32 = jnp.float32
    return pl.pallas_call(
        _node_body,
        out_shape=(
            jax.ShapeDtypeStruct((N_TOT, HID), f32),
            jax.ShapeDtypeStruct((N_TOT, 4), f32),
            jax.ShapeDtypeStruct((NP_PAD, HID), f32),
            jax.ShapeDtypeStruct((NP_PAD, HID), f32),
        ),
    )(h, pos, agg2, dp2, w1h, w1g, b1n, w2n, b2n, ew1a, ew1b, eb1)


def _final_body(hl, posl, offl, vw, vb, pos_out, v_out):
    pos_out[...] = posl[...] + offl[...]
    v_out[...] = jnp.dot(hl[...], vw[...]) + vb[...]


def _run_final(hl, posl, offl, vw, vb):
    f32 = jnp.float32
    return pl.pallas_call(
        _final_body,
        out_shape=(
            jax.ShapeDtypeStruct((N_LIG, 4), f32),
            jax.ShapeDtypeStruct((N_LIG, NUM_CLASSES), f32),
        ),
    )(hl, posl, offl, vw, vb)


def _pack_pos(pos):
    return jnp.reshape(jnp.pad(pos, ((0, NP_PAD - N_TOT), (0, 0))),
                       (NP_PAD * 4,))


def kernel(protein_pos, protein_v, batch_protein, init_ligand_pos,
           init_ligand_v, batch_ligand, time_step, edge_index,
           W_protein, b_protein, W_ligand, b_ligand,
           edge_w1, edge_b1, edge_w2, edge_b2,
           node_w1, node_b1, node_w2, node_b2,
           coord_w, coord_b, v_out_w, v_out_b):
    f32 = jnp.float32
    i32 = jnp.int32

    ppos4 = jnp.pad(protein_pos.astype(f32), ((0, 0), (0, 1)))
    lpos4 = jnp.pad(init_ligand_pos.astype(f32), ((0, 0), (0, 1)))
    bp_col = batch_protein.astype(i32).reshape(N_PROT, 1)
    bp_row = batch_protein.astype(i32).reshape(1, N_PROT)
    bl_col = batch_ligand.astype(i32).reshape(N_LIG, 1)
    bl_row = batch_ligand.astype(i32).reshape(1, N_LIG)
    t_col = time_step.astype(f32).reshape(NUM_GRAPHS, 1)

    wp = jnp.pad(W_protein, ((0, 0), (0, 1)))
    bpb = jnp.pad(b_protein, (0, 1)).reshape(1, HID)
    wlv = jnp.pad(W_ligand[:NUM_CLASSES], ((0, 0), (0, 1)))
    wlt = jnp.pad(W_ligand[NUM_CLASSES:], ((0, 0), (0, 1)))
    blb = jnp.concatenate([b_ligand, jnp.ones((1,), f32)]).reshape(1, HID)
    w1a = edge_w1[:HID]
    w1b = edge_w1[HID:2 * HID]
    w1c = edge_w1[2 * HID:]
    b1 = edge_b1.reshape(1, HID)
    b2 = edge_b2.reshape(1, HID)
    cb = coord_b.reshape(1, 1)
    w1h = node_w1[:HID]
    w1g = node_w1[HID:]
    b1n = node_b1.reshape(1, HID)
    b2n = node_b2.reshape(1, HID)
    vb = v_out_b.reshape(1, NUM_CLASSES)

    src = edge_index[0].astype(i32)
    dst = edge_index[1].astype(i32)
    pad_idx = N_TOT + jnp.arange(E_PAD - N_EDGE, dtype=i32) % (NP_PAD - N_TOT)
    srcm = jnp.concatenate([src, pad_idx]).reshape(NCHUNK * NW * NB, EB)
    dstm = jnp.concatenate([dst, pad_idx]).reshape(NCHUNK * NW * NB, EB)
    sm = [srcm[c * NW * NB:(c + 1) * NW * NB] for c in range(NCHUNK)]
    dm = [dstm[c * NW * NB:(c + 1) * NW * NB] for c in range(NCHUNK)]

    zagg = jnp.zeros((NP_PAD, HID), f32)
    zdp = jnp.zeros((DPF,), f32)

    h, pos, offl, ta, tb = _run_pre(
        ppos4, protein_v, bp_col, bp_row, lpos4, init_ligand_v, bl_col,
        bl_row, t_col, wp, bpb, wlv, wlt, blb, w1a, w1b, b1)

    for _ in range(NUM_LAYERS):
        posp = _pack_pos(pos)
        a0, rx0, ry0, rz0 = _run_gather(ta, tb, posp, sm[0], dm[0])
        a1, rx1, ry1, rz1 = _run_gather(ta, tb, posp, sm[1], dm[1])
        m0, wx0, wy0, wz0 = _run_edge(a0, rx0, ry0, rz0, w1c, edge_w2, b2,
                                      coord_w, cb)
        m1, wx1, wy1, wz1 = _run_edge(a1, rx1, ry1, rz1, w1c, edge_w2, b2,
                                      coord_w, cb)
        agg2, dpf2 = _run_scatter(m0, m1, wx0, wx1, wy0, wy1, wz0, wz1,
                                  dm[0], dm[1], zagg, zdp)
        dp2 = dpf2.reshape(NC, NP_PAD, 4)[:, :N_TOT]
        h, pos, ta, tb = _run_node(
            h, pos, agg2, dp2, w1h, w1g, b1n, node_w2, b2n, w1a, w1b, b1)

    posl4, pred_v = _run_final(
        h[N_PROT:], pos[N_PROT:], offl, v_out_w, vb)
    return posl4[:, :3], pred_v


# speedup vs baseline: 6.6052x; 1.0194x over previous
import functools
import math

import jax
import jax.numpy as jnp
from jax import lax
from jax.experimental import pallas as pl
from jax.experimental.pallas import tpu as pltpu
from jax.experimental.pallas import tpu_sc as plsc

N_PROT = 8000
N_LIG = 2000
N_TOT = 10000
N_EDGE = 320000
NUM_GRAPHS = 16
HID = 128
NUM_RBF = 16
R_MAX = 10.0
NUM_LAYERS = 2
NUM_CLASSES = 13

NC = 2
NS = 16
NW = NC * NS
EB = 128
NB = 16
EW = NB * EB
NCHUNK = 5
E_CH = NW * EW
E_PAD = NCHUNK * E_CH
NP_PAD = 10240
DPF = NP_PAD * 4
ROWS_PER_TILE = NP_PAD // NS
DPF_PER_TILE = DPF // NS


@functools.cache
def _mesh():
    return plsc.VectorSubcoreMesh(
        core_axis_name="c", subcore_axis_name="s",
        num_cores=NC, num_subcores=NS)


def _pre_body(ppos, pv, bp_col, bp_row, lpos, lv, bl_col, bl_row, t_col,
              wp, bpb, wlv, wlt, blb, w1a, w1b, b1,
              h_out, pos_out, offl_out, ta_out, tb_out):
    f32 = jnp.float32
    ppos_v = ppos[...]
    lpos_v = lpos[...]
    ohp_t = (lax.broadcasted_iota(jnp.int32, (NUM_GRAPHS, N_PROT), 0)
             == bp_row[...]).astype(f32)
    sums = jnp.dot(ohp_t, ppos_v)
    cnt = jnp.sum(ohp_t, axis=1, keepdims=True)
    mean = sums / jnp.maximum(cnt, 1.0)
    ohp = (lax.broadcasted_iota(jnp.int32, (N_PROT, NUM_GRAPHS), 1)
           == bp_col[...]).astype(f32)
    ohl = (lax.broadcasted_iota(jnp.int32, (N_LIG, NUM_GRAPHS), 1)
           == bl_col[...]).astype(f32)
    p_c = ppos_v - jnp.dot(ohp, mean)
    offl = jnp.dot(ohl, mean)
    l_c = lpos_v - offl
    half = 4
    freq = jnp.exp(lax.broadcasted_iota(jnp.int32, (1, half), 1).astype(f32)
                   * (-math.log(10000.0) / (half - 1)))
    ang = t_col[...] * freq
    te = jnp.concatenate([jnp.sin(ang), jnp.cos(ang)], axis=1)
    te_l = jnp.dot(ohl, te)
    h_p = jnp.dot(pv[...], wp[...]) + bpb[...]
    h_l = jnp.dot(lv[...], wlv[...]) + jnp.dot(te_l, wlt[...]) + blb[...]
    h = jnp.concatenate([h_p, h_l], axis=0)
    pos = jnp.concatenate([p_c, l_c], axis=0)
    h_out[...] = h
    pos_out[...] = pos
    offl_out[...] = offl
    zrows = jnp.zeros((NP_PAD - N_TOT, HID), f32)
    ta_out[...] = jnp.concatenate([jnp.dot(h, w1a[...]) + b1[...], zrows],
                                  axis=0)
    tb_out[...] = jnp.concatenate([jnp.dot(h, w1b[...]), zrows], axis=0)


def _run_pre(ppos4, pv, bp_col, bp_row, lpos4, lv, bl_col, bl_row, t_col,
             wp, bpb, wlv, wlt, blb, w1a, w1b, b1):
    f32 = jnp.float32
    return pl.pallas_call(
        _pre_body,
        out_shape=(
            jax.ShapeDtypeStruct((N_TOT, HID), f32),
            jax.ShapeDtypeStruct((N_TOT, 4), f32),
            jax.ShapeDtypeStruct((N_LIG, 4), f32),
            jax.ShapeDtypeStruct((NP_PAD, HID), f32),
            jax.ShapeDtypeStruct((NP_PAD, HID), f32),
        ),
    )(ppos4, pv, bp_col, bp_row, lpos4, lv, bl_col, bl_row, t_col,
      wp, bpb, wlv, wlt, blb, w1a, w1b, b1)


def _gather_body(ta_hbm, tb_hbm, posp_hbm, srcm_hbm, dstm_hbm,
                 as_out, rx_out, ry_out, rz_out,
                 idx_s, idx_d, posv,
                 ra0, rb0, rx0, ry0, rz0,
                 ra1, rb1, rx1, ry1, rz1,
                 gsem0, gsem1, osem0, osem1):
    cid = lax.axis_index("c")
    sid = lax.axis_index("s")
    wid = cid * NS + sid
    pltpu.sync_copy(srcm_hbm.at[pl.ds(wid * NB, NB)], idx_s)
    pltpu.sync_copy(dstm_hbm.at[pl.ds(wid * NB, NB)], idx_d)
    pltpu.sync_copy(posp_hbm, posv)

    sets = ((ra0, rb0, rx0, ry0, rz0, gsem0, osem0),
            (ra1, rb1, rx1, ry1, rz1, gsem1, osem1))

    def start(j, p):
        ra, rb = sets[p][0], sets[p][1]
        gsem = sets[p][5]
        pltpu.async_copy(ta_hbm.at[idx_s.at[j]], ra, gsem)
        pltpu.async_copy(tb_hbm.at[idx_d.at[j]], rb, gsem)

    def drain_out(j, p):
        ra, _, rxb, ryb, rzb, _, osem = sets[p]
        base = wid * EW + j * EB
        pltpu.make_async_copy(ra, as_out.at[pl.ds(base, EB)], osem).wait()
        pltpu.make_async_copy(rxb, rx_out.at[pl.ds(base, EB)], osem).wait()
        pltpu.make_async_copy(ryb, ry_out.at[pl.ds(base, EB)], osem).wait()
        pltpu.make_async_copy(rzb, rz_out.at[pl.ds(base, EB)], osem).wait()

    def process(j, p):
        ra, rb, rxb, ryb, rzb, gsem, osem = sets[p]

        def grp(gi, c2):
            s16 = idx_s[j, pl.ds(gi * 16, 16)]
            d16 = idx_d[j, pl.ds(gi * 16, 16)]
            fs = lax.shift_left(s16, 2)
            fd = lax.shift_left(d16, 2)
            for c, buf in ((0, rxb), (1, ryb), (2, rzb)):
                ps = plsc.load_gather(posv, [fs + c])
                pd = plsc.load_gather(posv, [fd + c])
                buf[pl.ds(gi * 16, 16)] = pd - ps
            return c2

        lax.fori_loop(0, EB // 16, grp, 0)
        pltpu.make_async_copy(ta_hbm.at[idx_s.at[j]], ra, gsem).wait()
        pltpu.make_async_copy(tb_hbm.at[idx_d.at[j]], rb, gsem).wait()

        @plsc.parallel_loop(0, EB * HID // 16, unroll=8)
        def _add(i):
            r = lax.shift_right_logical(i, 3)
            g = lax.bitwise_and(i, 7)
            ra[r, pl.ds(g * 16, 16)] += rb[r, pl.ds(g * 16, 16)]

        base = wid * EW + j * EB
        pltpu.async_copy(ra, as_out.at[pl.ds(base, EB)], osem)
        pltpu.async_copy(rxb, rx_out.at[pl.ds(base, EB)], osem)
        pltpu.async_copy(ryb, ry_out.at[pl.ds(base, EB)], osem)
        pltpu.async_copy(rzb, rz_out.at[pl.ds(base, EB)], osem)

    start(0, 0)

    def body(k, carry):
        j0 = 2 * k
        j1 = j0 + 1
        process(j0, 0)

        @pl.when(k > 0)
        def _():
            drain_out(j1 - 2, 1)

        start(j1, 1)
        process(j1, 1)
        drain_out(j0, 0)

        @pl.when(k < NB // 2 - 1)
        def _():
            start(j0 + 2, 0)

        return carry

    lax.fori_loop(0, NB // 2, body, 0)
    drain_out(NB - 1, 1)


def _run_gather(ta, tb, posp, srcm, dstm):
    f32 = jnp.float32
    return pl.kernel(
        _gather_body,
        out_type=(
            jax.ShapeDtypeStruct((E_CH, HID), f32),
            jax.ShapeDtypeStruct((E_CH,), f32),
            jax.ShapeDtypeStruct((E_CH,), f32),
            jax.ShapeDtypeStruct((E_CH,), f32),
        ),
        mesh=_mesh(),
        compiler_params=pltpu.CompilerParams(needs_layout_passes=False),
        scratch_types=[
            pltpu.VMEM((NB, EB), jnp.int32),
            pltpu.VMEM((NB, EB), jnp.int32),
            pltpu.VMEM((NP_PAD * 4,), f32),
            pltpu.VMEM((EB, HID), f32),
            pltpu.VMEM((EB, HID), f32),
            pltpu.VMEM((EB,), f32),
            pltpu.VMEM((EB,), f32),
            pltpu.VMEM((EB,), f32),
            pltpu.VMEM((EB, HID), f32),
            pltpu.VMEM((EB, HID), f32),
            pltpu.VMEM((EB,), f32),
            pltpu.VMEM((EB,), f32),
            pltpu.VMEM((EB,), f32),
            pltpu.SemaphoreType.DMA,
            pltpu.SemaphoreType.DMA,
            pltpu.SemaphoreType.DMA,
            pltpu.SemaphoreType.DMA,
        ],
    )(ta, tb, posp, srcm, dstm)


_EBLK = 2048


def _edge_body(asum, rx, ry, rz, w1c, w2, b2, cw, cb,
               m_out, wx_out, wy_out, wz_out):
    f32 = jnp.float32
    rxv = rx[...]
    ryv = ry[...]
    rzv = rz[...]
    d_flat = jnp.sqrt(rxv * rxv + ryv * ryv + rzv * rzv)
    d_row = d_flat.reshape(1, _EBLK)
    delta = R_MAX / (NUM_RBF - 1)
    cent_col = (lax.broadcasted_iota(jnp.int32, (NUM_RBF, 1), 0)
                .astype(f32) * delta)
    rbf_t = jnp.exp((-0.5 / (delta * delta)) * (d_row - cent_col) ** 2)
    contrib = lax.dot_general(rbf_t, w1c[...],
                              (((0,), (0,)), ((), ())))
    m1 = jnp.maximum(asum[...] + contrib, 0.0)
    m = jnp.maximum(jnp.dot(m1, w2[...]) + b2[...], 0.0)
    cw_row = lax.dot_general(cw[...], m,
                             (((0,), (1,)), ((), ())))
    w_row = (cw_row + cb[...]) / (d_row + 1.0)
    w_flat = jnp.squeeze(w_row, axis=0)
    m_out[...] = m
    wx_out[...] = rxv * w_flat
    wy_out[...] = ryv * w_flat
    wz_out[...] = rzv * w_flat


def _run_edge(asum, rx, ry, rz, w1c, w2, b2, cw, cb):
    f32 = jnp.float32
    nblk = E_CH // _EBLK
    full = lambda i: (0, 0)
    return pl.pallas_call(
        _edge_body,
        grid=(nblk,),
        in_specs=[
            pl.BlockSpec((_EBLK, HID), lambda i: (i, 0)),
            pl.BlockSpec((_EBLK,), lambda i: (i,)),
            pl.BlockSpec((_EBLK,), lambda i: (i,)),
            pl.BlockSpec((_EBLK,), lambda i: (i,)),
            pl.BlockSpec((NUM_RBF, HID), full),
            pl.BlockSpec((HID, HID), full),
            pl.BlockSpec((1, HID), full),
            pl.BlockSpec((HID, 1), full),
            pl.BlockSpec((1, 1), full),
        ],
        out_specs=[
            pl.BlockSpec((_EBLK, HID), lambda i: (i, 0)),
            pl.BlockSpec((_EBLK,), lambda i: (i,)),
            pl.BlockSpec((_EBLK,), lambda i: (i,)),
            pl.BlockSpec((_EBLK,), lambda i: (i,)),
        ],
        out_shape=(
            jax.ShapeDtypeStruct((E_CH, HID), f32),
            jax.ShapeDtypeStruct((E_CH,), f32),
            jax.ShapeDtypeStruct((E_CH,), f32),
            jax.ShapeDtypeStruct((E_CH,), f32),
        ),
    )(asum, rx, ry, rz, w1c, w2, b2, cw, cb)


def _scatter_body(m0_hbm, m1_hbm, m2_hbm, m3_hbm, m4_hbm,
                  wx0_hbm, wx1_hbm, wx2_hbm, wx3_hbm, wx4_hbm,
                  wy0_hbm, wy1_hbm, wy2_hbm, wy3_hbm, wy4_hbm,
                  wz0_hbm, wz1_hbm, wz2_hbm, wz3_hbm, wz4_hbm,
                  dstm0_hbm, dstm1_hbm, dstm2_hbm, dstm3_hbm, dstm4_hbm,
                  zagg_hbm, zdp_hbm,
                  agg_out, dpf_out,
                  idx_d, mr0, xb0, yb0, zb0, mr1, xb1, yb1, zb1,
                  fix, fiy, fiz, sh_agg, sh_dpf, sem0, sem1):
    cid = lax.axis_index("c")
    sid = lax.axis_index("s")
    wid = cid * NS + sid
    r0 = sid * ROWS_PER_TILE
    f0 = sid * DPF_PER_TILE
    pltpu.sync_copy(zagg_hbm.at[pl.ds(r0, ROWS_PER_TILE)],
                    sh_agg.at[pl.ds(r0, ROWS_PER_TILE)])
    pltpu.sync_copy(zdp_hbm.at[pl.ds(f0, DPF_PER_TILE)],
                    sh_dpf.at[pl.ds(f0, DPF_PER_TILE)])
    plsc.subcore_barrier()

    sets = ((mr0, xb0, yb0, zb0, sem0), (mr1, xb1, yb1, zb1, sem1))

    def one_chunk(m_hbm, wx_hbm, wy_hbm, wz_hbm, dstm_hbm):
        pltpu.sync_copy(dstm_hbm.at[pl.ds(wid * NB, NB)], idx_d)

        def start(j, p):
            mb, xb, yb, zb, sem = sets[p]
            base = wid * EW + j * EB
            pltpu.async_copy(m_hbm.at[pl.ds(base, EB)], mb, sem)
            pltpu.async_copy(wx_hbm.at[pl.ds(base, EB)], xb, sem)
            pltpu.async_copy(wy_hbm.at[pl.ds(base, EB)], yb, sem)
            pltpu.async_copy(wz_hbm.at[pl.ds(base, EB)], zb, sem)

        def process(j, p):
            mb, xb, yb, zb, sem = sets[p]

            def grp(gi, c2):
                d16 = idx_d[j, pl.ds(gi * 16, 16)]
                f = lax.shift_left(d16, 2)
                fix[pl.ds(gi * 16, 16)] = f
                fiy[pl.ds(gi * 16, 16)] = f + 1
                fiz[pl.ds(gi * 16, 16)] = f + 2
                return c2

            lax.fori_loop(0, EB // 16, grp, 0)
            base = wid * EW + j * EB
            pltpu.make_async_copy(m_hbm.at[pl.ds(base, EB)], mb, sem).wait()
            pltpu.make_async_copy(wx_hbm.at[pl.ds(base, EB)], xb, sem).wait()
            pltpu.make_async_copy(wy_hbm.at[pl.ds(base, EB)], yb, sem).wait()
            pltpu.make_async_copy(wz_hbm.at[pl.ds(base, EB)], zb, sem).wait()
            pltpu.sync_copy(mb, sh_agg.at[idx_d.at[j]], add=True)
            pltpu.sync_copy(xb, sh_dpf.at[fix], add=True)
            pltpu.sync_copy(yb, sh_dpf.at[fiy], add=True)
            pltpu.sync_copy(zb, sh_dpf.at[fiz], add=True)

        start(0, 0)

        def body(k, carry):
            j0 = 2 * k
            j1 = j0 + 1
            start(j1, 1)
            process(j0, 0)

            @pl.when(k < NB // 2 - 1)
            def _():
                start(j0 + 2, 0)

            process(j1, 1)
            return carry

        lax.fori_loop(0, NB // 2, body, 0)

    one_chunk(m0_hbm, wx0_hbm, wy0_hbm, wz0_hbm, dstm0_hbm)
    one_chunk(m1_hbm, wx1_hbm, wy1_hbm, wz1_hbm, dstm1_hbm)
    one_chunk(m2_hbm, wx2_hbm, wy2_hbm, wz2_hbm, dstm2_hbm)
    one_chunk(m3_hbm, wx3_hbm, wy3_hbm, wz3_hbm, dstm3_hbm)
    one_chunk(m4_hbm, wx4_hbm, wy4_hbm, wz4_hbm, dstm4_hbm)
    plsc.subcore_barrier()
    pltpu.sync_copy(sh_agg.at[pl.ds(r0, ROWS_PER_TILE)],
                    agg_out.at[cid, pl.ds(r0, ROWS_PER_TILE)])
    pltpu.sync_copy(sh_dpf.at[pl.ds(f0, DPF_PER_TILE)],
                    dpf_out.at[cid, pl.ds(f0, DPF_PER_TILE)])


def _run_scatter(ms, wxs, wys, wzs, dms, zagg, zdp):
    f32 = jnp.float32
    return pl.kernel(
        _scatter_body,
        out_type=(
            jax.ShapeDtypeStruct((NC, NP_PAD, HID), f32),
            jax.ShapeDtypeStruct((NC, DPF), f32),
        ),
        mesh=_mesh(),
        compiler_params=pltpu.CompilerParams(needs_layout_passes=False),
        scratch_types=[
            pltpu.VMEM((NB, EB), jnp.int32),
            pltpu.VMEM((EB, HID), f32),
            pltpu.VMEM((EB,), f32),
            pltpu.VMEM((EB,), f32),
            pltpu.VMEM((EB,), f32),
            pltpu.VMEM((EB, HID), f32),
            pltpu.VMEM((EB,), f32),
            pltpu.VMEM((EB,), f32),
            pltpu.VMEM((EB,), f32),
            pltpu.VMEM((EB,), jnp.int32),
            pltpu.VMEM((EB,), jnp.int32),
            pltpu.VMEM((EB,), jnp.int32),
            pltpu.VMEM_SHARED((NP_PAD, HID), f32),
            pltpu.VMEM_SHARED((DPF,), f32),
            pltpu.SemaphoreType.DMA,
            pltpu.SemaphoreType.DMA,
        ],
    )(*ms, *wxs, *wys, *wzs, *dms, zagg, zdp)


def _node_body(h, pos, agg2, dp2, w1h, w1g, b1n, w2n, b2n, ew1a, ew1b, eb1,
               h_out, pos_out, ta_out, tb_out):
    f32 = jnp.float32
    hv = h[...]
    a2 = agg2[...]
    agg = a2[0, :N_TOT] + a2[1, :N_TOT]
    u = jnp.maximum(jnp.dot(hv, w1h[...]) + jnp.dot(agg, w1g[...]) + b1n[...],
                    0.0)
    hn = hv + jnp.dot(u, w2n[...]) + b2n[...]
    d2v = dp2[...]
    dp = d2v[0] + d2v[1]
    mask = (lax.broadcasted_iota(jnp.int32, (N_TOT, 1), 0)
            >= N_PROT).astype(f32)
    posn = pos[...] + dp * mask
    h_out[...] = hn
    pos_out[...] = posn
    zrows = jnp.zeros((NP_PAD - N_TOT, HID), f32)
    ta_out[...] = jnp.concatenate([jnp.dot(hn, ew1a[...]) + eb1[...], zrows],
                                  axis=0)
    tb_out[...] = jnp.concatenate([jnp.dot(hn, ew1b[...]), zrows], axis=0)


def _run_node(h, pos, agg2, dp2, w1h, w1g, b1n, w2n, b2n, ew1a, ew1b, eb1):
    f32 = jnp.float32
    return pl.pallas_call(
        _node_body,
        out_shape=(
            jax.ShapeDtypeStruct((N_TOT, HID), f32),
            jax.ShapeDtypeStruct((N_TOT, 4), f32),
            jax.ShapeDtypeStruct((NP_PAD, HID), f32),
            jax.ShapeDtypeStruct((NP_PAD, HID), f32),
        ),
    )(h, pos, agg2, dp2, w1h, w1g, b1n, w2n, b2n, ew1a, ew1b, eb1)


def _final_body(hl, posl, offl, vw, vb, pos_out, v_out):
    pos_out[...] = posl[...] + offl[...]
    v_out[...] = jnp.dot(hl[...], vw[...]) + vb[...]


def _run_final(hl, posl, offl, vw, vb):
    f32 = jnp.float32
    return pl.pallas_call(
        _final_body,
        out_shape=(
            jax.ShapeDtypeStruct((N_LIG, 4), f32),
            jax.ShapeDtypeStruct((N_LIG, NUM_CLASSES), f32),
        ),
    )(hl, posl, offl, vw, vb)


def _pack_pos(pos):
    return jnp.reshape(jnp.pad(pos, ((0, NP_PAD - N_TOT), (0, 0))),
                       (NP_PAD * 4,))


def kernel(protein_pos, protein_v, batch_protein, init_ligand_pos,
           init_ligand_v, batch_ligand, time_step, edge_index,
           W_protein, b_protein, W_ligand, b_ligand,
           edge_w1, edge_b1, edge_w2, edge_b2,
           node_w1, node_b1, node_w2, node_b2,
           coord_w, coord_b, v_out_w, v_out_b):
    f32 = jnp.float32
    i32 = jnp.int32

    ppos4 = jnp.pad(protein_pos.astype(f32), ((0, 0), (0, 1)))
    lpos4 = jnp.pad(init_ligand_pos.astype(f32), ((0, 0), (0, 1)))
    bp_col = batch_protein.astype(i32).reshape(N_PROT, 1)
    bp_row = batch_protein.astype(i32).reshape(1, N_PROT)
    bl_col = batch_ligand.astype(i32).reshape(N_LIG, 1)
    bl_row = batch_ligand.astype(i32).reshape(1, N_LIG)
    t_col = time_step.astype(f32).reshape(NUM_GRAPHS, 1)

    wp = jnp.pad(W_protein, ((0, 0), (0, 1)))
    bpb = jnp.pad(b_protein, (0, 1)).reshape(1, HID)
    wlv = jnp.pad(W_ligand[:NUM_CLASSES], ((0, 0), (0, 1)))
    wlt = jnp.pad(W_ligand[NUM_CLASSES:], ((0, 0), (0, 1)))
    blb = jnp.concatenate([b_ligand, jnp.ones((1,), f32)]).reshape(1, HID)
    w1a = edge_w1[:HID]
    w1b = edge_w1[HID:2 * HID]
    w1c = edge_w1[2 * HID:]
    b1 = edge_b1.reshape(1, HID)
    b2 = edge_b2.reshape(1, HID)
    cb = coord_b.reshape(1, 1)
    w1h = node_w1[:HID]
    w1g = node_w1[HID:]
    b1n = node_b1.reshape(1, HID)
    b2n = node_b2.reshape(1, HID)
    vb = v_out_b.reshape(1, NUM_CLASSES)

    src = edge_index[0].astype(i32)
    dst = edge_index[1].astype(i32)
    pad_idx = N_TOT + jnp.arange(E_PAD - N_EDGE, dtype=i32) % (NP_PAD - N_TOT)
    srcm = jnp.concatenate([src, pad_idx]).reshape(NCHUNK * NW * NB, EB)
    dstm = jnp.concatenate([dst, pad_idx]).reshape(NCHUNK * NW * NB, EB)
    sm = [srcm[c * NW * NB:(c + 1) * NW * NB] for c in range(NCHUNK)]
    dm = [dstm[c * NW * NB:(c + 1) * NW * NB] for c in range(NCHUNK)]

    zagg = jnp.zeros((NP_PAD, HID), f32)
    zdp = jnp.zeros((DPF,), f32)

    h, pos, offl, ta, tb = _run_pre(
        ppos4, protein_v, bp_col, bp_row, lpos4, init_ligand_v, bl_col,
        bl_row, t_col, wp, bpb, wlv, wlt, blb, w1a, w1b, b1)

    for _ in range(NUM_LAYERS):
        posp = _pack_pos(pos)
        gout = [_run_gather(ta, tb, posp, sm[c], dm[c])
                for c in range(NCHUNK)]
        eout = [_run_edge(g[0], g[1], g[2], g[3], w1c, edge_w2, b2,
                          coord_w, cb) for g in gout]
        agg2, dpf2 = _run_scatter([e[0] for e in eout], [e[1] for e in eout],
                                  [e[2] for e in eout], [e[3] for e in eout],
                                  dm, zagg, zdp)
        dp2 = dpf2.reshape(NC, NP_PAD, 4)[:, :N_TOT]
        h, pos, ta, tb = _run_node(
            h, pos, agg2, dp2, w1h, w1g, b1n, node_w2, b2n, w1a, w1b, b1)

    posl4, pred_v = _run_final(
        h[N_PROT:], pos[N_PROT:], offl, v_out_w, vb)
    return posl4[:, :3], pred_v
